# Initial kernel scaffold; baseline (speedup 1.0000x reference)
import jax
import jax.numpy as jnp
from jax.experimental import pallas as pl


def kernel(features, edge_index, url, gat_W, attn_l, attn_r, gat_b, res_W, ln1_g, ln1_b, Wq, bq, Wk, bk, Wv, bv, fc_W, fc_b, ln2_g, ln2_b, out_W, out_b):
    raise NotImplementedError("write your pallas kernel here")


# speedup vs baseline: 28.8179x; 28.8179x over previous
import jax
import jax.numpy as jnp
from jax import lax
from jax.experimental import pallas as pl
from jax.experimental.pallas import tpu as pltpu
from jax.experimental.pallas import tpu_sc as plsc

N = 10000
E = 160000
FS = 128
H = 8
D = 64
HD = H * D
B = 1024
FW = HD + 128

NC = 2
NS = 16
NW = NC * NS
EC = E // NW
KMAX = 5072
CH = 64
AR = 5056
OWN = 32
PCAP = 5120
BPW = B // NW

_f32 = jnp.float32
_i32 = jnp.int32


def _tc1_body(x_ref, gw_ref, wl_ref, wr_ref, feat_ref, er_ref):
    f = jnp.dot(x_ref[...], gw_ref[...], preferred_element_type=_f32)
    feat_ref[:, pl.ds(0, HD)] = f
    feat_ref[:, pl.ds(HD, 16)] = jnp.dot(f, wl_ref[...],
                                         preferred_element_type=_f32)
    feat_ref[:, pl.ds(HD + 16, FW - HD - 16)] = jnp.zeros(
        (f.shape[0], FW - HD - 16), _f32)
    er_ref[:, pl.ds(0, 16)] = jnp.dot(f, wr_ref[...],
                                      preferred_element_type=_f32)
    er_ref[:, pl.ds(16, 112)] = jnp.zeros((f.shape[0], 112), _f32)


def _tc1(features, gat_W, Wl, Wr):
    R = 1000
    return pl.pallas_call(
        _tc1_body,
        grid=(N // R,),
        in_specs=[
            pl.BlockSpec((R, FS), lambda i: (i, 0)),
            pl.BlockSpec((FS, HD), lambda i: (0, 0)),
            pl.BlockSpec((HD, 16), lambda i: (0, 0)),
            pl.BlockSpec((HD, 16), lambda i: (0, 0)),
        ],
        out_specs=[
            pl.BlockSpec((R, FW), lambda i: (i, 0)),
            pl.BlockSpec((R, 128), lambda i: (i, 0)),
        ],
        out_shape=[
            jax.ShapeDtypeStruct((N, FW), _f32),
            jax.ShapeDtypeStruct((N, 128), _f32),
        ],
    )(features, gat_W, Wl, Wr)


def _mesh():
    return plsc.VectorSubcoreMesh(
        core_axis_name="c", subcore_axis_name="s",
        num_cores=NC, num_subcores=NS)


_SC_PARAMS = dict(compiler_params=pltpu.CompilerParams(
    needs_layout_passes=False))


def _sc_filter(src, dst, ids):
    k = pl.kernel(
        _sc_filter_body,
        out_type=(
            jax.ShapeDtypeStruct((NW * AR,), _i32),
            jax.ShapeDtypeStruct((NW * AR,), _i32),
            jax.ShapeDtypeStruct((NW * AR,), _i32),
            jax.ShapeDtypeStruct((NW * 16,), _i32),
            jax.ShapeDtypeStruct((B,), _i32),
        ),
        mesh=_mesh(),
        scratch_types=[
            pltpu.VMEM((N,), _i32),
            pltpu.VMEM((B,), _i32),
            pltpu.VMEM((KMAX,), _i32),
            pltpu.VMEM((KMAX,), _i32),
            pltpu.VMEM((KMAX,), _i32),
            pltpu.VMEM((BPW,), _i32),
            pltpu.VMEM((16,), _i32),
        ],
        **_SC_PARAMS,
    )
    return k(src, dst, ids)


def _sc_filter_body(src_hbm, dst_hbm, ids_hbm,
                    asrc_hbm, adst_hbm, aslot_hbm, cnt_hbm, rep_hbm,
                    pos, idsb, sbuf, dbuf, slist, repb, cntb):
    c = lax.axis_index("c")
    s = lax.axis_index("s")
    w = c * NS + s
    iota16 = jnp.arange(16, dtype=_i32)
    zero16i = jnp.zeros((16,), _i32)

    m1 = jnp.full((16,), -1, _i32)
    def _ip(i, _):
        pos[pl.ds(i * 16, 16)] = m1
        return 0
    lax.fori_loop(0, N // 16, _ip, 0)
    pltpu.sync_copy(ids_hbm, idsb)
    def _sp(j, _):
        idx = idsb[pl.ds(j * 16, 16)]
        plsc.store_scatter(pos, [idx], iota16 + j * 16)
        return 0
    lax.fori_loop(0, B // 16, _sp, 0)

    base = w * EC
    pltpu.sync_copy(src_hbm.at[pl.ds(base, EC)], sbuf.at[pl.ds(0, EC)])
    pltpu.sync_copy(dst_hbm.at[pl.ds(base, EC)], dbuf.at[pl.ds(0, EC)])
    plsc.store_scatter(sbuf, [EC + iota16], zero16i)
    plsc.store_scatter(dbuf, [EC + iota16], zero16i)

    def _flt(i, cnt):
        d16 = dbuf[pl.ds(i * 16, 16)]
        s16 = sbuf[pl.ds(i * 16, 16)]
        p16 = plsc.load_gather(pos, [d16])
        m = (p16 >= 0) & (iota16 < (EC - i * 16))
        plsc.store_compressed(sbuf.at[pl.ds(cnt, 16)], s16, mask=m)
        plsc.store_compressed(dbuf.at[pl.ds(cnt, 16)], d16, mask=m)
        plsc.store_compressed(slist.at[pl.ds(cnt, 16)], p16, mask=m)
        return cnt + jnp.sum(m.astype(_i32))
    kept = lax.fori_loop(0, (EC + 15) // 16, _flt, 0)

    dummy = jnp.full((16,), B, _i32)
    def _pad(j, _):
        idx = kept + j * 16 + iota16
        plsc.store_scatter(sbuf, [idx], zero16i)
        plsc.store_scatter(dbuf, [idx], zero16i)
        plsc.store_scatter(slist, [idx], dummy)
        return 0
    lax.fori_loop(0, CH // 16, _pad, 0)

    def _rep(j, _):
        idx = idsb[pl.ds(w * BPW + j * 16, 16)]
        repb[pl.ds(j * 16, 16)] = plsc.load_gather(pos, [idx])
        return 0
    lax.fori_loop(0, BPW // 16, _rep, 0)
    pltpu.sync_copy(repb, rep_hbm.at[pl.ds(w * BPW, BPW)])

    padded = lax.shift_left(lax.shift_right_logical(kept + (CH - 1), 6), 6)
    cntb[pl.ds(0, 16)] = jnp.zeros((16,), _i32) + padded
    pltpu.sync_copy(cntb, cnt_hbm.at[pl.ds(w * 16, 16)])
    nchunks = lax.shift_right_logical(padded, 6)
    def _pub(ci, _):
        off = ci * CH
        pltpu.sync_copy(sbuf.at[pl.ds(off, CH)],
                        asrc_hbm.at[pl.ds(w * AR + off, CH)])
        pltpu.sync_copy(dbuf.at[pl.ds(off, CH)],
                        adst_hbm.at[pl.ds(w * AR + off, CH)])
        pltpu.sync_copy(slist.at[pl.ds(off, CH)],
                        aslot_hbm.at[pl.ds(w * AR + off, CH)])
        return 0
    lax.fori_loop(0, nchunks, _pub, 0)


def _sc_accumulate(asrc, adst, aslot, cnts, feat, er):
    k = pl.kernel(
        _sc_accumulate_body,
        out_type=jax.ShapeDtypeStruct((B, FW), _f32),
        mesh=_mesh(),
        scratch_types=[
            pltpu.VMEM((NW * 16,), _i32),
            pltpu.VMEM((CH,), _i32),
            pltpu.VMEM((CH,), _i32),
            pltpu.VMEM((CH,), _i32),
            pltpu.VMEM((PCAP,), _i32),
            pltpu.VMEM((PCAP,), _i32),
            pltpu.VMEM((PCAP,), _i32),
            pltpu.VMEM((CH, FW), _f32),
            pltpu.VMEM((CH, 128), _f32),
            pltpu.VMEM((OWN + 8, FW), _f32),
        ],
        **_SC_PARAMS,
    )
    return k(asrc, adst, aslot, cnts, feat, er)


def _sc_accumulate_body(asrc_hbm, adst_hbm, aslot_hbm, cnt_hbm,
                        feat_hbm, er_hbm, u_hbm,
                        cbuf, tsrc, tdst, tslot, psrc, pdst, pslot,
                        featb, erb, u_loc):
    c = lax.axis_index("c")
    s = lax.axis_index("s")
    w = c * NS + s
    iota16 = jnp.arange(16, dtype=_i32)
    zero16f = jnp.zeros((16,), _f32)
    zero16i = jnp.zeros((16,), _i32)

    def _z(j, _):
        for q in range(FW // 16):
            u_loc[j, pl.ds(q * 16, 16)] = zero16f
        return 0
    lax.fori_loop(0, OWN + 8, _z, 0)

    pltpu.sync_copy(cnt_hbm, cbuf)

    def _process(off):
        pltpu.sync_copy(feat_hbm.at[psrc.at[pl.ds(off, CH)]], featb)
        pltpu.sync_copy(er_hbm.at[pdst.at[pl.ds(off, CH)]], erb)
        def _ee(j, _):
            e = featb[j, pl.ds(HD, 16)] + erb[j, pl.ds(0, 16)]
            e = jnp.where(e > 0.0, e, 0.2 * e)
            featb[j, pl.ds(HD, 16)] = jnp.exp(e)
            return 0
        lax.fori_loop(0, CH, _ee, 0)
        def _acc(g, _):
            row16 = pslot[pl.ds(off + g * 16, 16)]
            for k in range(16):
                j = g * 16 + k
                r = row16[k]
                eerow = featb[j, pl.ds(HD, 16)]
                for h in range(H):
                    sc = eerow[h]
                    for q in range(D // 16):
                        sl = pl.ds(h * D + q * 16, 16)
                        plsc.addupdate(u_loc.at[r, sl], featb[j, sl] * sc)
                plsc.addupdate(u_loc.at[r, pl.ds(HD, 16)], eerow)
            return 0
        lax.fori_loop(0, CH // 16, _acc, 0)

    def _tile(t, pend):
        cnt = cbuf[pl.ds(t * 16, 16)][0]
        nch = lax.shift_right_logical(cnt, 6)
        def _chunk(ci, pend):
            aoff = t * AR + ci * CH
            pltpu.sync_copy(asrc_hbm.at[pl.ds(aoff, CH)], tsrc)
            pltpu.sync_copy(adst_hbm.at[pl.ds(aoff, CH)], tdst)
            pltpu.sync_copy(aslot_hbm.at[pl.ds(aoff, CH)], tslot)
            for g in range(CH // 16):
                sl16 = tslot[pl.ds(g * 16, 16)]
                m = lax.shift_right_logical(sl16, 5) == w
                plsc.store_compressed(psrc.at[pl.ds(pend, 16)],
                                      tsrc[pl.ds(g * 16, 16)], mask=m)
                plsc.store_compressed(pdst.at[pl.ds(pend, 16)],
                                      tdst[pl.ds(g * 16, 16)], mask=m)
                plsc.store_compressed(pslot.at[pl.ds(pend, 16)],
                                      sl16 - OWN * w, mask=m)
                pend = pend + jnp.sum(m.astype(_i32))
            return pend
        pend = lax.fori_loop(0, nch, _chunk, pend)
        nd = lax.shift_right_logical(pend, 6)
        def _drain(dci, _):
            _process(dci * CH)
            return 0
        lax.fori_loop(0, nd, _drain, 0)
        rem_base = nd * CH
        for k in range(CH // 16):
            v0 = psrc[pl.ds(rem_base + k * 16, 16)]
            v1 = pdst[pl.ds(rem_base + k * 16, 16)]
            v2 = pslot[pl.ds(rem_base + k * 16, 16)]
            psrc[pl.ds(k * 16, 16)] = v0
            pdst[pl.ds(k * 16, 16)] = v1
            pslot[pl.ds(k * 16, 16)] = v2
        return pend - rem_base

    pend = lax.fori_loop(0, NW, _tile, 0)

    trash = jnp.full((16,), OWN, _i32)
    def _pad(j, _):
        idx = pend + j * 16 + iota16
        plsc.store_scatter(psrc, [idx], zero16i)
        plsc.store_scatter(pdst, [idx], zero16i)
        plsc.store_scatter(pslot, [idx], trash)
        return 0
    lax.fori_loop(0, CH // 16, _pad, 0)
    ntail = lax.shift_right_logical(pend + (CH - 1), 6)
    def _tail(dci, _):
        _process(dci * CH)
        return 0
    lax.fori_loop(0, ntail, _tail, 0)

    pltpu.sync_copy(u_loc.at[pl.ds(0, OWN)], u_hbm.at[pl.ds(w * OWN, OWN)])


def _sc_finalize(u, rep, features, ids):
    k = pl.kernel(
        _sc_finalize_body,
        out_type=(
            jax.ShapeDtypeStruct((B, FW), _f32),
            jax.ShapeDtypeStruct((B, FS), _f32),
        ),
        mesh=_mesh(),
        scratch_types=[
            pltpu.VMEM((BPW,), _i32),
            pltpu.VMEM((BPW, FW), _f32),
            pltpu.VMEM((BPW,), _i32),
            pltpu.VMEM((BPW, FS), _f32),
        ],
        **_SC_PARAMS,
    )
    return k(u, rep, features, ids)


def _sc_finalize_body(u_hbm, rep_hbm, features_hbm, ids_hbm,
                      sel_hbm, fsel_hbm, repb, ub, idsb, fb):
    c = lax.axis_index("c")
    s = lax.axis_index("s")
    w = c * NS + s
    pltpu.sync_copy(rep_hbm.at[pl.ds(w * BPW, BPW)], repb)
    pltpu.sync_copy(u_hbm.at[repb], ub)
    pltpu.sync_copy(ub, sel_hbm.at[pl.ds(w * BPW, BPW)])
    pltpu.sync_copy(ids_hbm.at[pl.ds(w * BPW, BPW)], idsb)
    pltpu.sync_copy(features_hbm.at[idsb], fb)
    pltpu.sync_copy(fb, fsel_hbm.at[pl.ds(w * BPW, BPW)])


def _tc3_body(sel_ref, fsel_ref, rw_ref, gb_ref, g1_ref, b1_ref,
              wq_ref, bq_ref, wk_ref, bk_ref, wv_ref, bv_ref,
              fw_ref, fb_ref, g2_ref, b2_ref, ow_ref, ob_ref,
              g8_ref, e8_ref, out_ref):
    g8 = g8_ref[...]
    e8 = e8_ref[...]
    den = jnp.dot(sel_ref[:, pl.ds(HD, H)], e8,
                  preferred_element_type=_f32) + 1e-9
    agg = sel_ref[:, pl.ds(0, HD)] / den
    x0 = agg + jnp.dot(fsel_ref[...], rw_ref[...],
                       preferred_element_type=_f32) + gb_ref[...]
    g8n = g8 * (1.0 / D)
    mu = jnp.dot(x0, g8n, preferred_element_type=_f32)
    xc = x0 - jnp.dot(mu, e8, preferred_element_type=_f32)
    var = jnp.dot(xc * xc, g8n, preferred_element_type=_f32)
    inv = jax.lax.rsqrt(var + 1e-6)
    x = xc * jnp.dot(inv, e8, preferred_element_type=_f32) * g1_ref[...] + b1_ref[...]

    qs, ks, vs, rsd = [], [], [], []
    for l in range(H):
        xl = x[:, l * D:(l + 1) * D]
        rsd.append(xl)
        qs.append(jnp.dot(xl, wq_ref[...], preferred_element_type=_f32) + bq_ref[...])
        ks.append(jnp.dot(xl, wk_ref[...], preferred_element_type=_f32) + bk_ref[...])
        vs.append(jnp.dot(xl, wv_ref[...], preferred_element_type=_f32) + bv_ref[...])

    acc = None
    scale = 1.0 / (D ** 0.5)
    for l in range(H):
        s_lm = [jnp.dot(qs[l] * ks[m], g8, preferred_element_type=_f32) * scale
                for m in range(H)]
        mx = s_lm[0]
        for m in range(1, H):
            mx = jnp.maximum(mx, s_lm[m])
        ex = [jnp.exp(sv - mx) for sv in s_lm]
        ssum = ex[0]
        for m in range(1, H):
            ssum = ssum + ex[m]
        rs = 1.0 / ssum
        o_l = None
        for m in range(H):
            t = jnp.dot(ex[m] * rs, e8, preferred_element_type=_f32) * vs[m]
            o_l = t if o_l is None else o_l + t
        f_l = jnp.dot(o_l, fw_ref[...], preferred_element_type=_f32) + fb_ref[...] + rsd[l]
        mu2 = jnp.mean(f_l, axis=1, keepdims=True)
        d2 = f_l - mu2
        var2 = jnp.mean(d2 * d2, axis=1, keepdims=True)
        z_l = d2 * jax.lax.rsqrt(var2 + 1e-5) * g2_ref[...] + b2_ref[...]
        acc = z_l if acc is None else acc + z_l
    out_ref[...] = jnp.dot(acc, ow_ref[...], preferred_element_type=_f32) + ob_ref[...]


def _tc3(sel, fsel, res_W, gat_b, g1, b1, Wq, bq, Wk, bk, Wv, bv,
         fc_W, fc_b, g2, b2, out_W, out_b, G8, E8):
    Bb = 128
    full = lambda shape: pl.BlockSpec(shape, lambda i: tuple(0 for _ in shape))
    return pl.pallas_call(
        _tc3_body,
        grid=(B // Bb,),
        in_specs=[
            pl.BlockSpec((Bb, FW), lambda i: (i, 0)),
            pl.BlockSpec((Bb, FS), lambda i: (i, 0)),
            full((FS, HD)), full((1, HD)), full((1, HD)), full((1, HD)),
            full((D, HD)), full((1, HD)), full((D, HD)), full((1, HD)),
            full((D, HD)), full((1, HD)),
            full((HD, D)), full((1, D)), full((1, D)), full((1, D)),
            full((D, 2)), full((1, 2)),
            full((HD, H)), full((H, HD)),
        ],
        out_specs=pl.BlockSpec((Bb, 2), lambda i: (i, 0)),
        out_shape=jax.ShapeDtypeStruct((B, 2), _f32),
    )(sel, fsel, res_W, gat_b, g1, b1, Wq, bq, Wk, bk, Wv, bv,
      fc_W, fc_b, g2, b2, out_W, out_b, G8, E8)


def kernel(features, edge_index, url, gat_W, attn_l, attn_r, gat_b, res_W,
           ln1_g, ln1_b, Wq, bq, Wk, bk, Wv, bv, fc_W, fc_b,
           ln2_g, ln2_b, out_W, out_b):
    ids = (url - 1).astype(_i32)
    src = edge_index[0]
    dst = edge_index[1]

    rows = jnp.arange(HD)
    cols = rows // D
    Wl = jnp.zeros((HD, 16), _f32).at[rows, cols].set(attn_l.reshape(-1))
    Wr = jnp.zeros((HD, 16), _f32).at[rows, cols].set(attn_r.reshape(-1))
    G8 = jnp.zeros((HD, H), _f32).at[rows, cols].set(1.0)
    E8 = G8.T

    feat, er = _tc1(features, gat_W, Wl, Wr)
    asrc, adst, aslot, cnts, rep = _sc_filter(src, dst, ids)
    u = _sc_accumulate(asrc, adst, aslot, cnts, feat, er)
    sel, fsel = _sc_finalize(u, rep, features, ids)

    logits = _tc3(
        sel, fsel, res_W, gat_b.reshape(1, HD),
        jnp.tile(ln1_g, H).reshape(1, HD), jnp.tile(ln1_b, H).reshape(1, HD),
        Wq, bq.reshape(1, HD), Wk, bk.reshape(1, HD), Wv, bv.reshape(1, HD),
        fc_W, fc_b.reshape(1, D), ln2_g.reshape(1, D), ln2_b.reshape(1, D),
        out_W, out_b.reshape(1, 2), G8, E8)
    return logits


# speedup vs baseline: 48.4463x; 1.6811x over previous
import jax
import jax.numpy as jnp
from jax import lax
from jax.experimental import pallas as pl
from jax.experimental.pallas import tpu as pltpu
from jax.experimental.pallas import tpu_sc as plsc

N = 10000
E = 160000
FS = 128
H = 8
D = 64
HD = H * D
B = 1024
FW = HD + 128

NC = 2
NS = 16
NW = NC * NS
EC = E // NW
KMAX = 5072
CH = 64
AR = 5056
OWN = 32
PCAP = 5120
BPW = B // NW

_f32 = jnp.float32
_i32 = jnp.int32


def _tc1_body(x_ref, gw_ref, wl_ref, wr_ref, feat_ref, er_ref):
    f = jnp.dot(x_ref[...], gw_ref[...], preferred_element_type=_f32)
    feat_ref[:, pl.ds(0, HD)] = f
    feat_ref[:, pl.ds(HD, 16)] = jnp.dot(f, wl_ref[...],
                                         preferred_element_type=_f32)
    feat_ref[:, pl.ds(HD + 16, FW - HD - 16)] = jnp.zeros(
        (f.shape[0], FW - HD - 16), _f32)
    er_ref[:, pl.ds(0, 16)] = jnp.dot(f, wr_ref[...],
                                      preferred_element_type=_f32)
    er_ref[:, pl.ds(16, 112)] = jnp.zeros((f.shape[0], 112), _f32)


def _tc1(features, gat_W, Wl, Wr):
    R = 1000
    return pl.pallas_call(
        _tc1_body,
        grid=(N // R,),
        in_specs=[
            pl.BlockSpec((R, FS), lambda i: (i, 0)),
            pl.BlockSpec((FS, HD), lambda i: (0, 0)),
            pl.BlockSpec((HD, 16), lambda i: (0, 0)),
            pl.BlockSpec((HD, 16), lambda i: (0, 0)),
        ],
        out_specs=[
            pl.BlockSpec((R, FW), lambda i: (i, 0)),
            pl.BlockSpec((R, 128), lambda i: (i, 0)),
        ],
        out_shape=[
            jax.ShapeDtypeStruct((N, FW), _f32),
            jax.ShapeDtypeStruct((N, 128), _f32),
        ],
    )(features, gat_W, Wl, Wr)


def _mesh():
    return plsc.VectorSubcoreMesh(
        core_axis_name="c", subcore_axis_name="s",
        num_cores=NC, num_subcores=NS)


_SC_PARAMS = dict(compiler_params=pltpu.CompilerParams(
    needs_layout_passes=False))


def _sc_filter(src, dst, ids):
    k = pl.kernel(
        _sc_filter_body,
        out_type=(
            jax.ShapeDtypeStruct((NW * AR,), _i32),
            jax.ShapeDtypeStruct((NW * AR,), _i32),
            jax.ShapeDtypeStruct((NW * AR,), _i32),
            jax.ShapeDtypeStruct((NW * 16,), _i32),
            jax.ShapeDtypeStruct((B,), _i32),
        ),
        mesh=_mesh(),
        scratch_types=[
            pltpu.VMEM((N,), _i32),
            pltpu.VMEM((B,), _i32),
            pltpu.VMEM((KMAX,), _i32),
            pltpu.VMEM((KMAX,), _i32),
            pltpu.VMEM((KMAX,), _i32),
            pltpu.VMEM((BPW,), _i32),
            pltpu.VMEM((16,), _i32),
        ],
        **_SC_PARAMS,
    )
    return k(src, dst, ids)


def _sc_filter_body(src_hbm, dst_hbm, ids_hbm,
                    asrc_hbm, adst_hbm, aslot_hbm, cnt_hbm, rep_hbm,
                    pos, idsb, sbuf, dbuf, slist, repb, cntb):
    c = lax.axis_index("c")
    s = lax.axis_index("s")
    w = c * NS + s
    iota16 = jnp.arange(16, dtype=_i32)
    zero16i = jnp.zeros((16,), _i32)

    m1 = jnp.full((16,), -1, _i32)
    def _ip(i, _):
        pos[pl.ds(i * 16, 16)] = m1
        return 0
    lax.fori_loop(0, N // 16, _ip, 0)
    pltpu.sync_copy(ids_hbm, idsb)
    def _sp(j, _):
        idx = idsb[pl.ds(j * 16, 16)]
        plsc.store_scatter(pos, [idx], iota16 + j * 16)
        return 0
    lax.fori_loop(0, B // 16, _sp, 0)

    base = w * EC
    pltpu.sync_copy(src_hbm.at[pl.ds(base, EC)], sbuf.at[pl.ds(0, EC)])
    pltpu.sync_copy(dst_hbm.at[pl.ds(base, EC)], dbuf.at[pl.ds(0, EC)])
    plsc.store_scatter(sbuf, [EC + iota16], zero16i)
    plsc.store_scatter(dbuf, [EC + iota16], zero16i)

    def _flt(i, cnt):
        d16 = dbuf[pl.ds(i * 16, 16)]
        s16 = sbuf[pl.ds(i * 16, 16)]
        p16 = plsc.load_gather(pos, [d16])
        m = (p16 >= 0) & (iota16 < (EC - i * 16))
        plsc.store_compressed(sbuf.at[pl.ds(cnt, 16)], s16, mask=m)
        plsc.store_compressed(dbuf.at[pl.ds(cnt, 16)], d16, mask=m)
        plsc.store_compressed(slist.at[pl.ds(cnt, 16)], p16, mask=m)
        return cnt + jnp.sum(m.astype(_i32))
    kept = lax.fori_loop(0, (EC + 15) // 16, _flt, 0)

    dummy = jnp.full((16,), B, _i32)
    def _pad(j, _):
        idx = kept + j * 16 + iota16
        plsc.store_scatter(sbuf, [idx], zero16i)
        plsc.store_scatter(dbuf, [idx], zero16i)
        plsc.store_scatter(slist, [idx], dummy)
        return 0
    lax.fori_loop(0, CH // 16, _pad, 0)

    def _rep(j, _):
        idx = idsb[pl.ds(w * BPW + j * 16, 16)]
        repb[pl.ds(j * 16, 16)] = plsc.load_gather(pos, [idx])
        return 0
    lax.fori_loop(0, BPW // 16, _rep, 0)
    pltpu.sync_copy(repb, rep_hbm.at[pl.ds(w * BPW, BPW)])

    padded = lax.shift_left(lax.shift_right_logical(kept + (CH - 1), 6), 6)
    cntb[pl.ds(0, 16)] = jnp.zeros((16,), _i32) + padded
    pltpu.sync_copy(cntb, cnt_hbm.at[pl.ds(w * 16, 16)])
    nchunks = lax.shift_right_logical(padded, 6)
    def _pub(ci, _):
        off = ci * CH
        pltpu.sync_copy(sbuf.at[pl.ds(off, CH)],
                        asrc_hbm.at[pl.ds(w * AR + off, CH)])
        pltpu.sync_copy(dbuf.at[pl.ds(off, CH)],
                        adst_hbm.at[pl.ds(w * AR + off, CH)])
        pltpu.sync_copy(slist.at[pl.ds(off, CH)],
                        aslot_hbm.at[pl.ds(w * AR + off, CH)])
        return 0
    lax.fori_loop(0, nchunks, _pub, 0)


CB = 1024


def _sc_accumulate(asrc, adst, aslot, cnts, feat, er):
    k = pl.kernel(
        _sc_accumulate_body,
        out_type=jax.ShapeDtypeStruct((B, FW), _f32),
        mesh=_mesh(),
        scratch_types=[
            pltpu.VMEM((NW * 16,), _i32),
            pltpu.VMEM((2 * CB,), _i32),
            pltpu.VMEM((2 * CB,), _i32),
            pltpu.VMEM((2 * CB,), _i32),
            pltpu.VMEM((PCAP,), _i32),
            pltpu.VMEM((PCAP,), _i32),
            pltpu.VMEM((PCAP,), _i32),
            pltpu.VMEM((CH, FW), _f32),
            pltpu.VMEM((CH, 128), _f32),
            pltpu.VMEM((OWN + 8, FW), _f32),
            pltpu.SemaphoreType.DMA,
            pltpu.SemaphoreType.DMA,
        ],
        **_SC_PARAMS,
    )
    return k(asrc, adst, aslot, cnts, feat, er)


def _sc_accumulate_body(asrc_hbm, adst_hbm, aslot_hbm, cnt_hbm,
                        feat_hbm, er_hbm, u_hbm,
                        cbuf, tsrc, tdst, tslot, psrc, pdst, pslot,
                        featb, erb, u_loc, asem, gsem):
    c = lax.axis_index("c")
    s = lax.axis_index("s")
    w = c * NS + s
    iota16 = jnp.arange(16, dtype=_i32)
    zero16f = jnp.zeros((16,), _f32)
    zero16i = jnp.zeros((16,), _i32)

    def _z(j, _):
        for q in range(FW // 16):
            u_loc[j, pl.ds(q * 16, 16)] = zero16f
        return 0
    lax.fori_loop(0, OWN + 8, _z, 0)

    pltpu.sync_copy(cnt_hbm, cbuf)

    def _process(off):
        d1 = pltpu.async_copy(feat_hbm.at[psrc.at[pl.ds(off, CH)]], featb,
                              gsem)
        d2 = pltpu.async_copy(er_hbm.at[pdst.at[pl.ds(off, CH)]], erb, gsem)
        d1.wait()
        d2.wait()
        def _ee(j, _):
            e = featb[j, pl.ds(HD, 16)] + erb[j, pl.ds(0, 16)]
            e = jnp.where(e > 0.0, e, 0.2 * e)
            featb[j, pl.ds(HD, 16)] = jnp.exp(e)
            return 0
        lax.fori_loop(0, CH, _ee, 0)
        def _acc(g, _):
            row16 = pslot[pl.ds(off + g * 16, 16)]
            for k in range(16):
                j = g * 16 + k
                r = row16[k]
                eerow = featb[j, pl.ds(HD, 16)]
                for h in range(H):
                    sc = eerow[h]
                    for q in range(D // 16):
                        sl = pl.ds(h * D + q * 16, 16)
                        plsc.addupdate(u_loc.at[r, sl], featb[j, sl] * sc)
                plsc.addupdate(u_loc.at[r, pl.ds(HD, 16)], eerow)
            return 0
        lax.fori_loop(0, CH // 16, _acc, 0)

    def _issue(t, parity):
        pltpu.async_copy(asrc_hbm.at[pl.ds(t * AR, CB)],
                         tsrc.at[pl.ds(parity * CB, CB)], asem)
        pltpu.async_copy(adst_hbm.at[pl.ds(t * AR, CB)],
                         tdst.at[pl.ds(parity * CB, CB)], asem)
        pltpu.async_copy(aslot_hbm.at[pl.ds(t * AR, CB)],
                         tslot.at[pl.ds(parity * CB, CB)], asem)

    def _wait(parity):
        pltpu.make_async_copy(asrc_hbm.at[pl.ds(0, CB)],
                              tsrc.at[pl.ds(parity * CB, CB)], asem).wait()
        pltpu.make_async_copy(adst_hbm.at[pl.ds(0, CB)],
                              tdst.at[pl.ds(parity * CB, CB)], asem).wait()
        pltpu.make_async_copy(aslot_hbm.at[pl.ds(0, CB)],
                              tslot.at[pl.ds(parity * CB, CB)], asem).wait()

    def _filter_append(boff, lim, lo, pend):
        def _grp(g, pend):
            sl16 = tslot[pl.ds(boff + g * 16, 16)]
            valid = (lo + g * 16 + iota16) < lim
            m = (lax.shift_right_logical(sl16, 5) == w) & valid
            plsc.store_compressed(psrc.at[pl.ds(pend, 16)],
                                  tsrc[pl.ds(boff + g * 16, 16)], mask=m)
            plsc.store_compressed(pdst.at[pl.ds(pend, 16)],
                                  tdst[pl.ds(boff + g * 16, 16)], mask=m)
            plsc.store_compressed(pslot.at[pl.ds(pend, 16)],
                                  sl16 - OWN * w, mask=m)
            return pend + jnp.sum(m.astype(_i32))
        ngrp = lax.shift_right_logical(
            jnp.minimum(lim - lo, CB) + 15, 4)
        return lax.fori_loop(0, ngrp, _grp, pend)

    _issue(0, 0)

    def _tile(t, pend):
        parity = lax.rem(t, 2)
        _wait(parity)
        @pl.when(t + 1 < NW)
        def _():
            _issue(t + 1, 1 - parity)
        cnt = cbuf[pl.ds(t * 16, 16)][0]
        pend = _filter_append(parity * CB, cnt, 0, pend)
        nex = lax.shift_right_logical(
            jnp.maximum(cnt, CB) - CB + (CB - 1), 10)
        def _extra(ec, pend):
            aoff = t * AR + CB + ec * CB
            pltpu.sync_copy(asrc_hbm.at[pl.ds(aoff, CB)],
                            tsrc.at[pl.ds(parity * CB, CB)])
            pltpu.sync_copy(adst_hbm.at[pl.ds(aoff, CB)],
                            tdst.at[pl.ds(parity * CB, CB)])
            pltpu.sync_copy(aslot_hbm.at[pl.ds(aoff, CB)],
                            tslot.at[pl.ds(parity * CB, CB)])
            return _filter_append(parity * CB, cnt, CB + ec * CB, pend)
        pend = lax.fori_loop(0, nex, _extra, pend)
        nd = lax.shift_right_logical(pend, 6)
        def _drain(dci, _):
            _process(dci * CH)
            return 0
        lax.fori_loop(0, nd, _drain, 0)
        rem_base = nd * CH
        for k in range(CH // 16):
            v0 = psrc[pl.ds(rem_base + k * 16, 16)]
            v1 = pdst[pl.ds(rem_base + k * 16, 16)]
            v2 = pslot[pl.ds(rem_base + k * 16, 16)]
            psrc[pl.ds(k * 16, 16)] = v0
            pdst[pl.ds(k * 16, 16)] = v1
            pslot[pl.ds(k * 16, 16)] = v2
        return pend - rem_base

    pend = lax.fori_loop(0, NW, _tile, 0)

    trash = jnp.full((16,), OWN, _i32)
    def _pad(j, _):
        idx = pend + j * 16 + iota16
        plsc.store_scatter(psrc, [idx], zero16i)
        plsc.store_scatter(pdst, [idx], zero16i)
        plsc.store_scatter(pslot, [idx], trash)
        return 0
    lax.fori_loop(0, CH // 16, _pad, 0)
    ntail = lax.shift_right_logical(pend + (CH - 1), 6)
    def _tail(dci, _):
        _process(dci * CH)
        return 0
    lax.fori_loop(0, ntail, _tail, 0)

    pltpu.sync_copy(u_loc.at[pl.ds(0, OWN)], u_hbm.at[pl.ds(w * OWN, OWN)])


def _sc_finalize(u, rep, features, ids):
    k = pl.kernel(
        _sc_finalize_body,
        out_type=(
            jax.ShapeDtypeStruct((B, FW), _f32),
            jax.ShapeDtypeStruct((B, FS), _f32),
        ),
        mesh=_mesh(),
        scratch_types=[
            pltpu.VMEM((BPW,), _i32),
            pltpu.VMEM((BPW, FW), _f32),
            pltpu.VMEM((BPW,), _i32),
            pltpu.VMEM((BPW, FS), _f32),
        ],
        **_SC_PARAMS,
    )
    return k(u, rep, features, ids)


def _sc_finalize_body(u_hbm, rep_hbm, features_hbm, ids_hbm,
                      sel_hbm, fsel_hbm, repb, ub, idsb, fb):
    c = lax.axis_index("c")
    s = lax.axis_index("s")
    w = c * NS + s
    pltpu.sync_copy(rep_hbm.at[pl.ds(w * BPW, BPW)], repb)
    pltpu.sync_copy(u_hbm.at[repb], ub)
    pltpu.sync_copy(ub, sel_hbm.at[pl.ds(w * BPW, BPW)])
    pltpu.sync_copy(ids_hbm.at[pl.ds(w * BPW, BPW)], idsb)
    pltpu.sync_copy(features_hbm.at[idsb], fb)
    pltpu.sync_copy(fb, fsel_hbm.at[pl.ds(w * BPW, BPW)])


def _tc3_body(sel_ref, fsel_ref, rw_ref, gb_ref, g1_ref, b1_ref,
              wq_ref, bq_ref, wk_ref, bk_ref, wv_ref, bv_ref,
              fw_ref, fb_ref, g2_ref, b2_ref, ow_ref, ob_ref,
              g8_ref, e8_ref, out_ref):
    g8 = g8_ref[...]
    e8 = e8_ref[...]
    den = jnp.dot(sel_ref[:, pl.ds(HD, H)], e8,
                  preferred_element_type=_f32) + 1e-9
    agg = sel_ref[:, pl.ds(0, HD)] / den
    x0 = agg + jnp.dot(fsel_ref[...], rw_ref[...],
                       preferred_element_type=_f32) + gb_ref[...]
    g8n = g8 * (1.0 / D)
    mu = jnp.dot(x0, g8n, preferred_element_type=_f32)
    xc = x0 - jnp.dot(mu, e8, preferred_element_type=_f32)
    var = jnp.dot(xc * xc, g8n, preferred_element_type=_f32)
    inv = jax.lax.rsqrt(var + 1e-6)
    x = xc * jnp.dot(inv, e8, preferred_element_type=_f32) * g1_ref[...] + b1_ref[...]

    qs, ks, vs, rsd = [], [], [], []
    for l in range(H):
        xl = x[:, l * D:(l + 1) * D]
        rsd.append(xl)
        qs.append(jnp.dot(xl, wq_ref[...], preferred_element_type=_f32) + bq_ref[...])
        ks.append(jnp.dot(xl, wk_ref[...], preferred_element_type=_f32) + bk_ref[...])
        vs.append(jnp.dot(xl, wv_ref[...], preferred_element_type=_f32) + bv_ref[...])

    acc = None
    scale = 1.0 / (D ** 0.5)
    for l in range(H):
        s_lm = [jnp.dot(qs[l] * ks[m], g8, preferred_element_type=_f32) * scale
                for m in range(H)]
        mx = s_lm[0]
        for m in range(1, H):
            mx = jnp.maximum(mx, s_lm[m])
        ex = [jnp.exp(sv - mx) for sv in s_lm]
        ssum = ex[0]
        for m in range(1, H):
            ssum = ssum + ex[m]
        rs = 1.0 / ssum
        o_l = None
        for m in range(H):
            t = jnp.dot(ex[m] * rs, e8, preferred_element_type=_f32) * vs[m]
            o_l = t if o_l is None else o_l + t
        f_l = jnp.dot(o_l, fw_ref[...], preferred_element_type=_f32) + fb_ref[...] + rsd[l]
        mu2 = jnp.mean(f_l, axis=1, keepdims=True)
        d2 = f_l - mu2
        var2 = jnp.mean(d2 * d2, axis=1, keepdims=True)
        z_l = d2 * jax.lax.rsqrt(var2 + 1e-5) * g2_ref[...] + b2_ref[...]
        acc = z_l if acc is None else acc + z_l
    out_ref[...] = jnp.dot(acc, ow_ref[...], preferred_element_type=_f32) + ob_ref[...]


def _tc3(sel, fsel, res_W, gat_b, g1, b1, Wq, bq, Wk, bk, Wv, bv,
         fc_W, fc_b, g2, b2, out_W, out_b, G8, E8):
    Bb = 128
    full = lambda shape: pl.BlockSpec(shape, lambda i: tuple(0 for _ in shape))
    return pl.pallas_call(
        _tc3_body,
        grid=(B // Bb,),
        in_specs=[
            pl.BlockSpec((Bb, FW), lambda i: (i, 0)),
            pl.BlockSpec((Bb, FS), lambda i: (i, 0)),
            full((FS, HD)), full((1, HD)), full((1, HD)), full((1, HD)),
            full((D, HD)), full((1, HD)), full((D, HD)), full((1, HD)),
            full((D, HD)), full((1, HD)),
            full((HD, D)), full((1, D)), full((1, D)), full((1, D)),
            full((D, 2)), full((1, 2)),
            full((HD, H)), full((H, HD)),
        ],
        out_specs=pl.BlockSpec((Bb, 2), lambda i: (i, 0)),
        out_shape=jax.ShapeDtypeStruct((B, 2), _f32),
    )(sel, fsel, res_W, gat_b, g1, b1, Wq, bq, Wk, bk, Wv, bv,
      fc_W, fc_b, g2, b2, out_W, out_b, G8, E8)


def kernel(features, edge_index, url, gat_W, attn_l, attn_r, gat_b, res_W,
           ln1_g, ln1_b, Wq, bq, Wk, bk, Wv, bv, fc_W, fc_b,
           ln2_g, ln2_b, out_W, out_b):
    ids = (url - 1).astype(_i32)
    src = edge_index[0]
    dst = edge_index[1]

    rows = jnp.arange(HD)
    cols = rows // D
    Wl = jnp.zeros((HD, 16), _f32).at[rows, cols].set(attn_l.reshape(-1))
    Wr = jnp.zeros((HD, 16), _f32).at[rows, cols].set(attn_r.reshape(-1))
    G8 = jnp.zeros((HD, H), _f32).at[rows, cols].set(1.0)
    E8 = G8.T

    feat, er = _tc1(features, gat_W, Wl, Wr)
    asrc, adst, aslot, cnts, rep = _sc_filter(src, dst, ids)
    u = _sc_accumulate(asrc, adst, aslot, cnts, feat, er)
    sel, fsel = _sc_finalize(u, rep, features, ids)

    logits = _tc3(
        sel, fsel, res_W, gat_b.reshape(1, HD),
        jnp.tile(ln1_g, H).reshape(1, HD), jnp.tile(ln1_b, H).reshape(1, HD),
        Wq, bq.reshape(1, HD), Wk, bk.reshape(1, HD), Wv, bv.reshape(1, HD),
        fc_W, fc_b.reshape(1, D), ln2_g.reshape(1, D), ln2_b.reshape(1, D),
        out_W, out_b.reshape(1, 2), G8, E8)
    return logits


# speedup vs baseline: 64.2205x; 1.3256x over previous
import jax
import jax.numpy as jnp
from jax import lax
from jax.experimental import pallas as pl
from jax.experimental.pallas import tpu as pltpu
from jax.experimental.pallas import tpu_sc as plsc

N = 10000
E = 160000
FS = 128
H = 8
D = 64
HD = H * D
B = 1024
FW = HD + 128

NC = 2
NS = 16
NW = NC * NS
EC = E // NW
KMAX = 5072
CH = 64
AR = 5056
OWN = 32
PCAP = 8192
BPW = B // NW

_f32 = jnp.float32
_i32 = jnp.int32


def _tc1_body(x_ref, gw_ref, wl_ref, wr_ref, feat_ref, er_ref):
    f = jnp.dot(x_ref[...], gw_ref[...], preferred_element_type=_f32)
    feat_ref[:, pl.ds(0, HD)] = f
    feat_ref[:, pl.ds(HD, 16)] = jnp.dot(f, wl_ref[...],
                                         preferred_element_type=_f32)
    feat_ref[:, pl.ds(HD + 16, FW - HD - 16)] = jnp.zeros(
        (f.shape[0], FW - HD - 16), _f32)
    er_ref[:, pl.ds(0, 16)] = jnp.dot(f, wr_ref[...],
                                      preferred_element_type=_f32)
    er_ref[:, pl.ds(16, 112)] = jnp.zeros((f.shape[0], 112), _f32)


def _tc1(features, gat_W, Wl, Wr):
    R = 1000
    return pl.pallas_call(
        _tc1_body,
        grid=(N // R,),
        in_specs=[
            pl.BlockSpec((R, FS), lambda i: (i, 0)),
            pl.BlockSpec((FS, HD), lambda i: (0, 0)),
            pl.BlockSpec((HD, 16), lambda i: (0, 0)),
            pl.BlockSpec((HD, 16), lambda i: (0, 0)),
        ],
        out_specs=[
            pl.BlockSpec((R, FW), lambda i: (i, 0)),
            pl.BlockSpec((R, 128), lambda i: (i, 0)),
        ],
        out_shape=[
            jax.ShapeDtypeStruct((N, FW), _f32),
            jax.ShapeDtypeStruct((N, 128), _f32),
        ],
    )(features, gat_W, Wl, Wr)


def _mesh():
    return plsc.VectorSubcoreMesh(
        core_axis_name="c", subcore_axis_name="s",
        num_cores=NC, num_subcores=NS)


_SC_PARAMS = dict(compiler_params=pltpu.CompilerParams(
    needs_layout_passes=False))


def _sc_filter(src, dst, ids, features):
    k = pl.kernel(
        _sc_filter_body,
        out_type=(
            jax.ShapeDtypeStruct((NW * AR,), _i32),
            jax.ShapeDtypeStruct((NW * AR,), _i32),
            jax.ShapeDtypeStruct((NW * AR,), _i32),
            jax.ShapeDtypeStruct((NW * 16,), _i32),
            jax.ShapeDtypeStruct((B,), _i32),
            jax.ShapeDtypeStruct((B, FS), _f32),
        ),
        mesh=_mesh(),
        scratch_types=[
            pltpu.VMEM((N,), _i32),
            pltpu.VMEM((B,), _i32),
            pltpu.VMEM((KMAX,), _i32),
            pltpu.VMEM((KMAX,), _i32),
            pltpu.VMEM((KMAX,), _i32),
            pltpu.VMEM((BPW,), _i32),
            pltpu.VMEM((16,), _i32),
            pltpu.VMEM((BPW, FS), _f32),
        ],
        **_SC_PARAMS,
    )
    return k(src, dst, ids, features)


def _sc_filter_body(src_hbm, dst_hbm, ids_hbm, features_hbm,
                    asrc_hbm, adst_hbm, aslot_hbm, cnt_hbm, rep_hbm, fsel_hbm,
                    pos, idsb, sbuf, dbuf, slist, repb, cntb, fb):
    c = lax.axis_index("c")
    s = lax.axis_index("s")
    w = c * NS + s
    iota16 = jnp.arange(16, dtype=_i32)
    zero16i = jnp.zeros((16,), _i32)

    m1 = jnp.full((16,), -1, _i32)
    def _ip(i, _):
        pos[pl.ds(i * 16, 16)] = m1
        return 0
    lax.fori_loop(0, N // 16, _ip, 0)
    pltpu.sync_copy(ids_hbm, idsb)
    def _sp(j, _):
        idx = idsb[pl.ds(j * 16, 16)]
        plsc.store_scatter(pos, [idx], iota16 + j * 16)
        return 0
    lax.fori_loop(0, B // 16, _sp, 0)

    base = w * EC
    pltpu.sync_copy(src_hbm.at[pl.ds(base, EC)], sbuf.at[pl.ds(0, EC)])
    pltpu.sync_copy(dst_hbm.at[pl.ds(base, EC)], dbuf.at[pl.ds(0, EC)])
    plsc.store_scatter(sbuf, [EC + iota16], zero16i)
    plsc.store_scatter(dbuf, [EC + iota16], zero16i)

    def _flt(i, cnt):
        d16 = dbuf[pl.ds(i * 16, 16)]
        s16 = sbuf[pl.ds(i * 16, 16)]
        p16 = plsc.load_gather(pos, [d16])
        m = (p16 >= 0) & (iota16 < (EC - i * 16))
        plsc.store_compressed(sbuf.at[pl.ds(cnt, 16)], s16, mask=m)
        plsc.store_compressed(dbuf.at[pl.ds(cnt, 16)], d16, mask=m)
        plsc.store_compressed(slist.at[pl.ds(cnt, 16)], p16, mask=m)
        return cnt + jnp.sum(m.astype(_i32))
    kept = lax.fori_loop(0, (EC + 15) // 16, _flt, 0)

    dummy = jnp.full((16,), B, _i32)
    def _pad(j, _):
        idx = kept + j * 16 + iota16
        plsc.store_scatter(sbuf, [idx], zero16i)
        plsc.store_scatter(dbuf, [idx], zero16i)
        plsc.store_scatter(slist, [idx], dummy)
        return 0
    lax.fori_loop(0, CH // 16, _pad, 0)

    def _rep(j, _):
        idx = idsb[pl.ds(w * BPW + j * 16, 16)]
        repb[pl.ds(j * 16, 16)] = plsc.load_gather(pos, [idx])
        return 0
    lax.fori_loop(0, BPW // 16, _rep, 0)
    pltpu.sync_copy(repb, rep_hbm.at[pl.ds(w * BPW, BPW)])

    pltpu.sync_copy(features_hbm.at[idsb.at[pl.ds(w * BPW, BPW)]], fb)
    pltpu.sync_copy(fb, fsel_hbm.at[pl.ds(w * BPW, BPW)])

    padded = lax.shift_left(lax.shift_right_logical(kept + (CH - 1), 6), 6)
    cntb[pl.ds(0, 16)] = jnp.zeros((16,), _i32) + padded
    pltpu.sync_copy(cntb, cnt_hbm.at[pl.ds(w * 16, 16)])
    nchunks = lax.shift_right_logical(padded, 6)
    def _pub(ci, _):
        off = ci * CH
        pltpu.sync_copy(sbuf.at[pl.ds(off, CH)],
                        asrc_hbm.at[pl.ds(w * AR + off, CH)])
        pltpu.sync_copy(dbuf.at[pl.ds(off, CH)],
                        adst_hbm.at[pl.ds(w * AR + off, CH)])
        pltpu.sync_copy(slist.at[pl.ds(off, CH)],
                        aslot_hbm.at[pl.ds(w * AR + off, CH)])
        return 0
    lax.fori_loop(0, nchunks, _pub, 0)


CB = 1024
PCH = 32


def _sc_accumulate(asrc, adst, aslot, cnts, feat, er):
    k = pl.kernel(
        _sc_accumulate_body,
        out_type=jax.ShapeDtypeStruct((B, FW), _f32),
        mesh=_mesh(),
        scratch_types=[
            pltpu.VMEM((NW * 16,), _i32),
            pltpu.VMEM((2 * CB,), _i32),
            pltpu.VMEM((2 * CB,), _i32),
            pltpu.VMEM((2 * CB,), _i32),
            pltpu.VMEM((PCAP,), _i32),
            pltpu.VMEM((PCAP,), _i32),
            pltpu.VMEM((PCAP,), _i32),
            pltpu.VMEM((2 * PCH, FW), _f32),
            pltpu.VMEM((2 * PCH, 128), _f32),
            pltpu.VMEM((OWN + 8, FW), _f32),
            pltpu.SemaphoreType.DMA,
            pltpu.SemaphoreType.DMA,
        ],
        **_SC_PARAMS,
    )
    return k(asrc, adst, aslot, cnts, feat, er)


def _sc_accumulate_body(asrc_hbm, adst_hbm, aslot_hbm, cnt_hbm,
                        feat_hbm, er_hbm, u_hbm,
                        cbuf, tsrc, tdst, tslot, psrc, pdst, pslot,
                        featb, erb, u_loc, asem, gsem):
    c = lax.axis_index("c")
    s = lax.axis_index("s")
    w = c * NS + s
    iota16 = jnp.arange(16, dtype=_i32)
    zero16f = jnp.zeros((16,), _f32)
    zero16i = jnp.zeros((16,), _i32)

    def _z(j, _):
        for q in range(FW // 16):
            u_loc[j, pl.ds(q * 16, 16)] = zero16f
        return 0
    lax.fori_loop(0, OWN + 8, _z, 0)

    pltpu.sync_copy(cnt_hbm, cbuf)

    def _gissue(off, parity):
        pltpu.async_copy(feat_hbm.at[psrc.at[pl.ds(off, PCH)]],
                         featb.at[pl.ds(parity * PCH, PCH)], gsem)
        pltpu.async_copy(er_hbm.at[pdst.at[pl.ds(off, PCH)]],
                         erb.at[pl.ds(parity * PCH, PCH)], gsem)

    def _gwait(parity):
        pltpu.make_async_copy(feat_hbm.at[pl.ds(0, PCH)],
                              featb.at[pl.ds(parity * PCH, PCH)], gsem).wait()
        pltpu.make_async_copy(er_hbm.at[pl.ds(0, PCH)],
                              erb.at[pl.ds(parity * PCH, PCH)], gsem).wait()

    def _compute(off, parity):
        base = parity * PCH
        def _ee(j, _):
            e = featb[base + j, pl.ds(HD, 16)] + erb[base + j, pl.ds(0, 16)]
            e = jnp.where(e > 0.0, e, 0.2 * e)
            featb[base + j, pl.ds(HD, 16)] = jnp.exp(e)
            return 0
        lax.fori_loop(0, PCH, _ee, 0)
        def _acc(g, _):
            row16 = pslot[pl.ds(off + g * 16, 16)]
            for k in range(16):
                j = base + g * 16 + k
                r = row16[k]
                eerow = featb[j, pl.ds(HD, 16)]
                for h in range(H):
                    sc = eerow[h]
                    for q in range(D // 16):
                        sl = pl.ds(h * D + q * 16, 16)
                        plsc.addupdate(u_loc.at[r, sl], featb[j, sl] * sc)
                plsc.addupdate(u_loc.at[r, pl.ds(HD, 16)], eerow)
            return 0
        lax.fori_loop(0, PCH // 16, _acc, 0)

    def _run(nproc):
        @pl.when(nproc > 0)
        def _():
            _gissue(0, 0)
            def _p(i, _):
                parity = lax.rem(i, 2)
                _gwait(parity)
                @pl.when(i + 1 < nproc)
                def _():
                    _gissue((i + 1) * PCH, 1 - parity)
                _compute(i * PCH, parity)
                return 0
            lax.fori_loop(0, nproc, _p, 0)

    def _issue(t, parity):
        pltpu.async_copy(asrc_hbm.at[pl.ds(t * AR, CB)],
                         tsrc.at[pl.ds(parity * CB, CB)], asem)
        pltpu.async_copy(adst_hbm.at[pl.ds(t * AR, CB)],
                         tdst.at[pl.ds(parity * CB, CB)], asem)
        pltpu.async_copy(aslot_hbm.at[pl.ds(t * AR, CB)],
                         tslot.at[pl.ds(parity * CB, CB)], asem)

    def _wait(parity):
        pltpu.make_async_copy(asrc_hbm.at[pl.ds(0, CB)],
                              tsrc.at[pl.ds(parity * CB, CB)], asem).wait()
        pltpu.make_async_copy(adst_hbm.at[pl.ds(0, CB)],
                              tdst.at[pl.ds(parity * CB, CB)], asem).wait()
        pltpu.make_async_copy(aslot_hbm.at[pl.ds(0, CB)],
                              tslot.at[pl.ds(parity * CB, CB)], asem).wait()

    def _filter_append(boff, lim, lo, pend):
        def _grp(g, pend):
            sl16 = tslot[pl.ds(boff + g * 16, 16)]
            valid = (lo + g * 16 + iota16) < lim
            m = (lax.shift_right_logical(sl16, 5) == w) & valid
            plsc.store_compressed(psrc.at[pl.ds(pend, 16)],
                                  tsrc[pl.ds(boff + g * 16, 16)], mask=m)
            plsc.store_compressed(pdst.at[pl.ds(pend, 16)],
                                  tdst[pl.ds(boff + g * 16, 16)], mask=m)
            plsc.store_compressed(pslot.at[pl.ds(pend, 16)],
                                  sl16 - OWN * w, mask=m)
            return pend + jnp.sum(m.astype(_i32))
        ngrp = lax.shift_right_logical(
            jnp.minimum(lim - lo, CB) + 15, 4)
        return lax.fori_loop(0, ngrp, _grp, pend)

    _issue(0, 0)

    def _tile(t, pend):
        parity = lax.rem(t, 2)
        _wait(parity)
        @pl.when(t + 1 < NW)
        def _():
            _issue(t + 1, 1 - parity)
        cnt = cbuf[pl.ds(t * 16, 16)][0]
        pend = _filter_append(parity * CB, cnt, 0, pend)
        nex = lax.shift_right_logical(
            jnp.maximum(cnt, CB) - CB + (CB - 1), 10)
        def _extra(ec, pend):
            aoff = t * AR + CB + ec * CB
            pltpu.sync_copy(asrc_hbm.at[pl.ds(aoff, CB)],
                            tsrc.at[pl.ds(parity * CB, CB)])
            pltpu.sync_copy(adst_hbm.at[pl.ds(aoff, CB)],
                            tdst.at[pl.ds(parity * CB, CB)])
            pltpu.sync_copy(aslot_hbm.at[pl.ds(aoff, CB)],
                            tslot.at[pl.ds(parity * CB, CB)])
            return _filter_append(parity * CB, cnt, CB + ec * CB, pend)
        pend = lax.fori_loop(0, nex, _extra, pend)
        nd = jnp.where(pend >= PCAP - AR,
                       lax.shift_right_logical(pend, 5), 0)
        _run(nd)
        rem_base = nd * PCH
        for k in range(PCH // 16):
            v0 = psrc[pl.ds(rem_base + k * 16, 16)]
            v1 = pdst[pl.ds(rem_base + k * 16, 16)]
            v2 = pslot[pl.ds(rem_base + k * 16, 16)]
            psrc[pl.ds(k * 16, 16)] = v0
            pdst[pl.ds(k * 16, 16)] = v1
            pslot[pl.ds(k * 16, 16)] = v2
        return pend - rem_base

    pend = lax.fori_loop(0, NW, _tile, 0)

    trash = jnp.full((16,), OWN, _i32)
    def _pad(j, _):
        idx = pend + j * 16 + iota16
        plsc.store_scatter(psrc, [idx], zero16i)
        plsc.store_scatter(pdst, [idx], zero16i)
        plsc.store_scatter(pslot, [idx], trash)
        return 0
    lax.fori_loop(0, PCH // 16, _pad, 0)
    _run(lax.shift_right_logical(pend + (PCH - 1), 5))

    pltpu.sync_copy(u_loc.at[pl.ds(0, OWN)], u_hbm.at[pl.ds(w * OWN, OWN)])


def _tc3_body(sel_ref, fsel_ref, rw_ref, gb_ref, g1_ref, b1_ref,
              wq_ref, bq_ref, wk_ref, bk_ref, wv_ref, bv_ref,
              fw_ref, fb_ref, g2_ref, b2_ref, ow_ref, ob_ref,
              g8_ref, e8_ref, out_ref):
    g8 = g8_ref[...]
    e8 = e8_ref[...]
    den = jnp.dot(sel_ref[:, pl.ds(HD, H)], e8,
                  preferred_element_type=_f32) + 1e-9
    agg = sel_ref[:, pl.ds(0, HD)] / den
    x0 = agg + jnp.dot(fsel_ref[...], rw_ref[...],
                       preferred_element_type=_f32) + gb_ref[...]
    g8n = g8 * (1.0 / D)
    mu = jnp.dot(x0, g8n, preferred_element_type=_f32)
    xc = x0 - jnp.dot(mu, e8, preferred_element_type=_f32)
    var = jnp.dot(xc * xc, g8n, preferred_element_type=_f32)
    inv = jax.lax.rsqrt(var + 1e-6)
    x = xc * jnp.dot(inv, e8, preferred_element_type=_f32) * g1_ref[...] + b1_ref[...]

    qs, ks, vs, rsd = [], [], [], []
    for l in range(H):
        xl = x[:, l * D:(l + 1) * D]
        rsd.append(xl)
        qs.append(jnp.dot(xl, wq_ref[...], preferred_element_type=_f32) + bq_ref[...])
        ks.append(jnp.dot(xl, wk_ref[...], preferred_element_type=_f32) + bk_ref[...])
        vs.append(jnp.dot(xl, wv_ref[...], preferred_element_type=_f32) + bv_ref[...])

    acc = None
    scale = 1.0 / (D ** 0.5)
    for l in range(H):
        s_lm = [jnp.dot(qs[l] * ks[m], g8, preferred_element_type=_f32) * scale
                for m in range(H)]
        mx = s_lm[0]
        for m in range(1, H):
            mx = jnp.maximum(mx, s_lm[m])
        ex = [jnp.exp(sv - mx) for sv in s_lm]
        ssum = ex[0]
        for m in range(1, H):
            ssum = ssum + ex[m]
        rs = 1.0 / ssum
        o_l = None
        for m in range(H):
            t = jnp.dot(ex[m] * rs, e8, preferred_element_type=_f32) * vs[m]
            o_l = t if o_l is None else o_l + t
        f_l = jnp.dot(o_l, fw_ref[...], preferred_element_type=_f32) + fb_ref[...] + rsd[l]
        mu2 = jnp.mean(f_l, axis=1, keepdims=True)
        d2 = f_l - mu2
        var2 = jnp.mean(d2 * d2, axis=1, keepdims=True)
        z_l = d2 * jax.lax.rsqrt(var2 + 1e-5) * g2_ref[...] + b2_ref[...]
        acc = z_l if acc is None else acc + z_l
    out_ref[...] = jnp.dot(acc, ow_ref[...], preferred_element_type=_f32) + ob_ref[...]


def _tc3(sel, fsel, res_W, gat_b, g1, b1, Wq, bq, Wk, bk, Wv, bv,
         fc_W, fc_b, g2, b2, out_W, out_b, G8, E8):
    Bb = 128
    full = lambda shape: pl.BlockSpec(shape, lambda i: tuple(0 for _ in shape))
    return pl.pallas_call(
        _tc3_body,
        grid=(B // Bb,),
        in_specs=[
            pl.BlockSpec((Bb, FW), lambda i: (i, 0)),
            pl.BlockSpec((Bb, FS), lambda i: (i, 0)),
            full((FS, HD)), full((1, HD)), full((1, HD)), full((1, HD)),
            full((D, HD)), full((1, HD)), full((D, HD)), full((1, HD)),
            full((D, HD)), full((1, HD)),
            full((HD, D)), full((1, D)), full((1, D)), full((1, D)),
            full((D, 2)), full((1, 2)),
            full((HD, H)), full((H, HD)),
        ],
        out_specs=pl.BlockSpec((Bb, 2), lambda i: (i, 0)),
        out_shape=jax.ShapeDtypeStruct((B, 2), _f32),
    )(sel, fsel, res_W, gat_b, g1, b1, Wq, bq, Wk, bk, Wv, bv,
      fc_W, fc_b, g2, b2, out_W, out_b, G8, E8)


def kernel(features, edge_index, url, gat_W, attn_l, attn_r, gat_b, res_W,
           ln1_g, ln1_b, Wq, bq, Wk, bk, Wv, bv, fc_W, fc_b,
           ln2_g, ln2_b, out_W, out_b):
    ids = (url - 1).astype(_i32)
    src = edge_index[0]
    dst = edge_index[1]

    rows = jnp.arange(HD)[:, None]
    cols16 = jnp.arange(16)[None, :]
    blk = (rows // D) == cols16
    Wl = jnp.where(blk, attn_l.reshape(-1)[:, None], 0.0)
    Wr = jnp.where(blk, attn_r.reshape(-1)[:, None], 0.0)
    G8 = jnp.where((rows // D) == jnp.arange(H)[None, :], 1.0, 0.0)
    E8 = G8.T

    feat, er = _tc1(features, gat_W, Wl, Wr)
    asrc, adst, aslot, cnts, rep, fsel = _sc_filter(src, dst, ids, features)
    u = _sc_accumulate(asrc, adst, aslot, cnts, feat, er)

    logits_slot = _tc3(
        u, fsel, res_W, gat_b.reshape(1, HD),
        jnp.tile(ln1_g, H).reshape(1, HD), jnp.tile(ln1_b, H).reshape(1, HD),
        Wq, bq.reshape(1, HD), Wk, bk.reshape(1, HD), Wv, bv.reshape(1, HD),
        fc_W, fc_b.reshape(1, D), ln2_g.reshape(1, D), ln2_b.reshape(1, D),
        out_W, out_b.reshape(1, 2), G8, E8)
    return logits_slot[rep]


# speedup vs baseline: 68.1289x; 1.0609x over previous
import jax
import jax.numpy as jnp
from jax import lax
from jax.experimental import pallas as pl
from jax.experimental.pallas import tpu as pltpu
from jax.experimental.pallas import tpu_sc as plsc

N = 10000
E = 160000
FS = 128
H = 8
D = 64
HD = H * D
B = 1024
FW = HD + 128

NC = 2
NS = 16
NW = NC * NS
EC = E // NW
KMAX = 5072
CH = 64
AR = 5056
OWN = 32
PCAP = 6144
BPW = B // NW

_f32 = jnp.float32
_i32 = jnp.int32


def _tc1_body(x_ref, gw_ref, wl_ref, wr_ref, feat_ref, er_ref):
    f = jnp.dot(x_ref[...], gw_ref[...], preferred_element_type=_f32)
    feat_ref[:, pl.ds(0, HD)] = f
    feat_ref[:, pl.ds(HD, 16)] = jnp.dot(f, wl_ref[...],
                                         preferred_element_type=_f32)
    feat_ref[:, pl.ds(HD + 16, FW - HD - 16)] = jnp.zeros(
        (f.shape[0], FW - HD - 16), _f32)
    er_ref[:, pl.ds(0, 16)] = jnp.dot(f, wr_ref[...],
                                      preferred_element_type=_f32)
    er_ref[:, pl.ds(16, 112)] = jnp.zeros((f.shape[0], 112), _f32)


def _tc1(features, gat_W, Wl, Wr):
    R = 1000
    return pl.pallas_call(
        _tc1_body,
        grid=(N // R,),
        in_specs=[
            pl.BlockSpec((R, FS), lambda i: (i, 0)),
            pl.BlockSpec((FS, HD), lambda i: (0, 0)),
            pl.BlockSpec((HD, 16), lambda i: (0, 0)),
            pl.BlockSpec((HD, 16), lambda i: (0, 0)),
        ],
        out_specs=[
            pl.BlockSpec((R, FW), lambda i: (i, 0)),
            pl.BlockSpec((R, 128), lambda i: (i, 0)),
        ],
        out_shape=[
            jax.ShapeDtypeStruct((N, FW), _f32),
            jax.ShapeDtypeStruct((N, 128), _f32),
        ],
    )(features, gat_W, Wl, Wr)


def _mesh():
    return plsc.VectorSubcoreMesh(
        core_axis_name="c", subcore_axis_name="s",
        num_cores=NC, num_subcores=NS)


_SC_PARAMS = dict(compiler_params=pltpu.CompilerParams(
    needs_layout_passes=False))


def _sc_filter(src, dst, ids, features):
    k = pl.kernel(
        _sc_filter_body,
        out_type=(
            jax.ShapeDtypeStruct((NW * AR,), _i32),
            jax.ShapeDtypeStruct((NW * AR,), _i32),
            jax.ShapeDtypeStruct((NW * AR,), _i32),
            jax.ShapeDtypeStruct((NW * 16,), _i32),
            jax.ShapeDtypeStruct((B,), _i32),
            jax.ShapeDtypeStruct((B, FS), _f32),
        ),
        mesh=_mesh(),
        scratch_types=[
            pltpu.VMEM((N,), _i32),
            pltpu.VMEM((B,), _i32),
            pltpu.VMEM((KMAX,), _i32),
            pltpu.VMEM((KMAX,), _i32),
            pltpu.VMEM((KMAX,), _i32),
            pltpu.VMEM((BPW,), _i32),
            pltpu.VMEM((16,), _i32),
            pltpu.VMEM((BPW, FS), _f32),
        ],
        **_SC_PARAMS,
    )
    return k(src, dst, ids, features)


def _sc_filter_body(src_hbm, dst_hbm, ids_hbm, features_hbm,
                    asrc_hbm, adst_hbm, aslot_hbm, cnt_hbm, rep_hbm, fsel_hbm,
                    pos, idsb, sbuf, dbuf, slist, repb, cntb, fb):
    c = lax.axis_index("c")
    s = lax.axis_index("s")
    w = c * NS + s
    iota16 = jnp.arange(16, dtype=_i32)
    zero16i = jnp.zeros((16,), _i32)

    m1 = jnp.full((16,), -1, _i32)
    def _ip(i, _):
        pos[pl.ds(i * 16, 16)] = m1
        return 0
    lax.fori_loop(0, N // 16, _ip, 0)
    pltpu.sync_copy(ids_hbm, idsb)
    def _sp(j, _):
        idx = idsb[pl.ds(j * 16, 16)]
        plsc.store_scatter(pos, [idx], iota16 + j * 16)
        return 0
    lax.fori_loop(0, B // 16, _sp, 0)

    base = w * EC
    pltpu.sync_copy(src_hbm.at[pl.ds(base, EC)], sbuf.at[pl.ds(0, EC)])
    pltpu.sync_copy(dst_hbm.at[pl.ds(base, EC)], dbuf.at[pl.ds(0, EC)])
    plsc.store_scatter(sbuf, [EC + iota16], zero16i)
    plsc.store_scatter(dbuf, [EC + iota16], zero16i)

    def _flt(i, cnt):
        d16 = dbuf[pl.ds(i * 16, 16)]
        s16 = sbuf[pl.ds(i * 16, 16)]
        p16 = plsc.load_gather(pos, [d16])
        m = (p16 >= 0) & (iota16 < (EC - i * 16))
        plsc.store_compressed(sbuf.at[pl.ds(cnt, 16)], s16, mask=m)
        plsc.store_compressed(dbuf.at[pl.ds(cnt, 16)], d16, mask=m)
        plsc.store_compressed(slist.at[pl.ds(cnt, 16)], p16, mask=m)
        return cnt + jnp.sum(m.astype(_i32))
    kept = lax.fori_loop(0, (EC + 15) // 16, _flt, 0)

    dummy = jnp.full((16,), B, _i32)
    def _pad(j, _):
        idx = kept + j * 16 + iota16
        plsc.store_scatter(sbuf, [idx], zero16i)
        plsc.store_scatter(dbuf, [idx], zero16i)
        plsc.store_scatter(slist, [idx], dummy)
        return 0
    lax.fori_loop(0, CH // 16, _pad, 0)

    def _rep(j, _):
        idx = idsb[pl.ds(w * BPW + j * 16, 16)]
        repb[pl.ds(j * 16, 16)] = plsc.load_gather(pos, [idx])
        return 0
    lax.fori_loop(0, BPW // 16, _rep, 0)
    pltpu.sync_copy(repb, rep_hbm.at[pl.ds(w * BPW, BPW)])

    pltpu.sync_copy(features_hbm.at[idsb.at[pl.ds(w * BPW, BPW)]], fb)
    pltpu.sync_copy(fb, fsel_hbm.at[pl.ds(w * BPW, BPW)])

    padded = lax.shift_left(lax.shift_right_logical(kept + (CH - 1), 6), 6)
    cntb[pl.ds(0, 16)] = jnp.zeros((16,), _i32) + padded
    pltpu.sync_copy(cntb, cnt_hbm.at[pl.ds(w * 16, 16)])
    nchunks = lax.shift_right_logical(padded, 6)
    def _pub(ci, _):
        off = ci * CH
        pltpu.sync_copy(sbuf.at[pl.ds(off, CH)],
                        asrc_hbm.at[pl.ds(w * AR + off, CH)])
        pltpu.sync_copy(dbuf.at[pl.ds(off, CH)],
                        adst_hbm.at[pl.ds(w * AR + off, CH)])
        pltpu.sync_copy(slist.at[pl.ds(off, CH)],
                        aslot_hbm.at[pl.ds(w * AR + off, CH)])
        return 0
    lax.fori_loop(0, nchunks, _pub, 0)


CB = 1024
PCH = 32
RING = 8


def _sc_accumulate(asrc, adst, aslot, cnts, feat, er, ids):
    k = pl.kernel(
        _sc_accumulate_body,
        out_type=jax.ShapeDtypeStruct((B, FW), _f32),
        mesh=_mesh(),
        scratch_types=[
            pltpu.VMEM((NW * 16,), _i32),
            pltpu.VMEM((RING * CB,), _i32),
            pltpu.VMEM((RING * CB,), _i32),
            pltpu.VMEM((RING * CB,), _i32),
            pltpu.VMEM((PCAP,), _i32),
            pltpu.VMEM((PCAP,), _i32),
            pltpu.VMEM((PCAP,), _i32),
            pltpu.VMEM((2 * PCH, FW), _f32),
            pltpu.VMEM((OWN + 8, 128), _f32),
            pltpu.VMEM((BPW,), _i32),
            pltpu.VMEM((OWN + 8, FW), _f32),
            pltpu.SemaphoreType.DMA,
            pltpu.SemaphoreType.DMA,
        ],
        **_SC_PARAMS,
    )
    return k(asrc, adst, aslot, cnts, feat, er, ids)


def _sc_accumulate_body(asrc_hbm, adst_hbm, aslot_hbm, cnt_hbm,
                        feat_hbm, er_hbm, ids_hbm, u_hbm,
                        cbuf, tsrc, tdst, tslot, psrc, pdst, pslot,
                        featb, er_loc, idsb, u_loc, asem, gsem):
    c = lax.axis_index("c")
    s = lax.axis_index("s")
    w = c * NS + s
    iota16 = jnp.arange(16, dtype=_i32)
    zero16f = jnp.zeros((16,), _f32)
    zero16i = jnp.zeros((16,), _i32)

    def _z(j, _):
        for q in range(FW // 16):
            u_loc[j, pl.ds(q * 16, 16)] = zero16f
        er_loc[j, pl.ds(0, 16)] = zero16f
        return 0
    lax.fori_loop(0, OWN + 8, _z, 0)

    pltpu.sync_copy(cnt_hbm, cbuf)

    pltpu.sync_copy(ids_hbm.at[pl.ds(w * OWN, OWN)], idsb)
    pltpu.sync_copy(er_hbm.at[idsb], er_loc.at[pl.ds(0, OWN)])

    def _gissue(off, parity):
        pltpu.async_copy(feat_hbm.at[psrc.at[pl.ds(off, PCH)]],
                         featb.at[pl.ds(parity * PCH, PCH)], gsem)

    def _gwait(parity):
        pltpu.make_async_copy(feat_hbm.at[pl.ds(0, PCH)],
                              featb.at[pl.ds(parity * PCH, PCH)], gsem).wait()

    def _compute(off, parity):
        base = parity * PCH
        def _acc(g, _):
            row16 = pslot[pl.ds(off + g * 16, 16)]
            for k in range(16):
                j = base + g * 16 + k
                r = row16[k]
                e = featb[j, pl.ds(HD, 16)] + er_loc[r, pl.ds(0, 16)]
                e = jnp.where(e > 0.0, e, 0.2 * e)
                eerow = jnp.exp(e)
                for h in range(H):
                    sc = eerow[h]
                    for q in range(D // 16):
                        sl = pl.ds(h * D + q * 16, 16)
                        plsc.addupdate(u_loc.at[r, sl], featb[j, sl] * sc)
                plsc.addupdate(u_loc.at[r, pl.ds(HD, 16)], eerow)
            return 0
        lax.fori_loop(0, PCH // 16, _acc, 0)

    def _run(nproc):
        @pl.when(nproc > 0)
        def _():
            _gissue(0, 0)
            def _p(i, _):
                parity = lax.rem(i, 2)
                _gwait(parity)
                @pl.when(i + 1 < nproc)
                def _():
                    _gissue((i + 1) * PCH, 1 - parity)
                _compute(i * PCH, parity)
                return 0
            lax.fori_loop(0, nproc, _p, 0)

    def _issue(t, parity):
        pltpu.async_copy(asrc_hbm.at[pl.ds(t * AR, CB)],
                         tsrc.at[pl.ds(parity * CB, CB)], asem)
        pltpu.async_copy(adst_hbm.at[pl.ds(t * AR, CB)],
                         tdst.at[pl.ds(parity * CB, CB)], asem)
        pltpu.async_copy(aslot_hbm.at[pl.ds(t * AR, CB)],
                         tslot.at[pl.ds(parity * CB, CB)], asem)

    def _wait(parity):
        pltpu.make_async_copy(asrc_hbm.at[pl.ds(0, CB)],
                              tsrc.at[pl.ds(parity * CB, CB)], asem).wait()
        pltpu.make_async_copy(adst_hbm.at[pl.ds(0, CB)],
                              tdst.at[pl.ds(parity * CB, CB)], asem).wait()
        pltpu.make_async_copy(aslot_hbm.at[pl.ds(0, CB)],
                              tslot.at[pl.ds(parity * CB, CB)], asem).wait()

    def _filter_append(boff, lim, lo, pend):
        def _grp(g, pend):
            sl16 = tslot[pl.ds(boff + g * 16, 16)]
            valid = (lo + g * 16 + iota16) < lim
            m = (lax.shift_right_logical(sl16, 5) == w) & valid
            plsc.store_compressed(psrc.at[pl.ds(pend, 16)],
                                  tsrc[pl.ds(boff + g * 16, 16)], mask=m)
            plsc.store_compressed(pdst.at[pl.ds(pend, 16)],
                                  tdst[pl.ds(boff + g * 16, 16)], mask=m)
            plsc.store_compressed(pslot.at[pl.ds(pend, 16)],
                                  sl16 - OWN * w, mask=m)
            return pend + jnp.sum(m.astype(_i32))
        ngrp = lax.shift_right_logical(
            jnp.minimum(lim - lo, CB) + 15, 4)
        return lax.fori_loop(0, ngrp, _grp, pend)

    for r in range(RING):
        _issue(r, r)

    def _tile(t, pend):
        slot = lax.rem(t, RING)
        _wait(slot)
        cnt = cbuf[pl.ds(t * 16, 16)][0]
        pend = _filter_append(slot * CB, cnt, 0, pend)
        nex = lax.shift_right_logical(
            jnp.maximum(cnt, CB) - CB + (CB - 1), 10)
        def _extra(ec, pend):
            aoff = t * AR + CB + ec * CB
            pltpu.sync_copy(asrc_hbm.at[pl.ds(aoff, CB)],
                            tsrc.at[pl.ds(slot * CB, CB)])
            pltpu.sync_copy(adst_hbm.at[pl.ds(aoff, CB)],
                            tdst.at[pl.ds(slot * CB, CB)])
            pltpu.sync_copy(aslot_hbm.at[pl.ds(aoff, CB)],
                            tslot.at[pl.ds(slot * CB, CB)])
            return _filter_append(slot * CB, cnt, CB + ec * CB, pend)
        pend = lax.fori_loop(0, nex, _extra, pend)
        @pl.when(t + RING < NW)
        def _():
            _issue(t + RING, slot)
        nd = jnp.where(pend >= PCAP - AR,
                       lax.shift_right_logical(pend, 5), 0)
        _run(nd)
        rem_base = nd * PCH
        for k in range(PCH // 16):
            v0 = psrc[pl.ds(rem_base + k * 16, 16)]
            v1 = pdst[pl.ds(rem_base + k * 16, 16)]
            v2 = pslot[pl.ds(rem_base + k * 16, 16)]
            psrc[pl.ds(k * 16, 16)] = v0
            pdst[pl.ds(k * 16, 16)] = v1
            pslot[pl.ds(k * 16, 16)] = v2
        return pend - rem_base

    pend = lax.fori_loop(0, NW, _tile, 0)

    trash = jnp.full((16,), OWN, _i32)
    def _pad(j, _):
        idx = pend + j * 16 + iota16
        plsc.store_scatter(psrc, [idx], zero16i)
        plsc.store_scatter(pdst, [idx], zero16i)
        plsc.store_scatter(pslot, [idx], trash)
        return 0
    lax.fori_loop(0, PCH // 16, _pad, 0)
    _run(lax.shift_right_logical(pend + (PCH - 1), 5))

    pltpu.sync_copy(u_loc.at[pl.ds(0, OWN)], u_hbm.at[pl.ds(w * OWN, OWN)])


def _tc3_body(sel_ref, fsel_ref, rw_ref, gb_ref, g1_ref, b1_ref,
              wq_ref, bq_ref, wk_ref, bk_ref, wv_ref, bv_ref,
              fw_ref, fb_ref, g2_ref, b2_ref, ow_ref, ob_ref,
              g8_ref, e8_ref, out_ref):
    g8 = g8_ref[...]
    e8 = e8_ref[...]
    den = jnp.dot(sel_ref[:, pl.ds(HD, H)], e8,
                  preferred_element_type=_f32) + 1e-9
    agg = sel_ref[:, pl.ds(0, HD)] / den
    x0 = agg + jnp.dot(fsel_ref[...], rw_ref[...],
                       preferred_element_type=_f32) + gb_ref[...]
    g8n = g8 * (1.0 / D)
    mu = jnp.dot(x0, g8n, preferred_element_type=_f32)
    xc = x0 - jnp.dot(mu, e8, preferred_element_type=_f32)
    var = jnp.dot(xc * xc, g8n, preferred_element_type=_f32)
    inv = jax.lax.rsqrt(var + 1e-6)
    x = xc * jnp.dot(inv, e8, preferred_element_type=_f32) * g1_ref[...] + b1_ref[...]

    qs, ks, vs, rsd = [], [], [], []
    for l in range(H):
        xl = x[:, l * D:(l + 1) * D]
        rsd.append(xl)
        qs.append(jnp.dot(xl, wq_ref[...], preferred_element_type=_f32) + bq_ref[...])
        ks.append(jnp.dot(xl, wk_ref[...], preferred_element_type=_f32) + bk_ref[...])
        vs.append(jnp.dot(xl, wv_ref[...], preferred_element_type=_f32) + bv_ref[...])

    acc = None
    scale = 1.0 / (D ** 0.5)
    for l in range(H):
        s_lm = [jnp.dot(qs[l] * ks[m], g8, preferred_element_type=_f32) * scale
                for m in range(H)]
        mx = s_lm[0]
        for m in range(1, H):
            mx = jnp.maximum(mx, s_lm[m])
        ex = [jnp.exp(sv - mx) for sv in s_lm]
        ssum = ex[0]
        for m in range(1, H):
            ssum = ssum + ex[m]
        rs = 1.0 / ssum
        o_l = None
        for m in range(H):
            t = jnp.dot(ex[m] * rs, e8, preferred_element_type=_f32) * vs[m]
            o_l = t if o_l is None else o_l + t
        f_l = jnp.dot(o_l, fw_ref[...], preferred_element_type=_f32) + fb_ref[...] + rsd[l]
        mu2 = jnp.mean(f_l, axis=1, keepdims=True)
        d2 = f_l - mu2
        var2 = jnp.mean(d2 * d2, axis=1, keepdims=True)
        z_l = d2 * jax.lax.rsqrt(var2 + 1e-5) * g2_ref[...] + b2_ref[...]
        acc = z_l if acc is None else acc + z_l
    out_ref[...] = jnp.dot(acc, ow_ref[...], preferred_element_type=_f32) + ob_ref[...]


def _tc3(sel, fsel, res_W, gat_b, g1, b1, Wq, bq, Wk, bk, Wv, bv,
         fc_W, fc_b, g2, b2, out_W, out_b, G8, E8):
    Bb = 128
    full = lambda shape: pl.BlockSpec(shape, lambda i: tuple(0 for _ in shape))
    return pl.pallas_call(
        _tc3_body,
        grid=(B // Bb,),
        in_specs=[
            pl.BlockSpec((Bb, FW), lambda i: (i, 0)),
            pl.BlockSpec((Bb, FS), lambda i: (i, 0)),
            full((FS, HD)), full((1, HD)), full((1, HD)), full((1, HD)),
            full((D, HD)), full((1, HD)), full((D, HD)), full((1, HD)),
            full((D, HD)), full((1, HD)),
            full((HD, D)), full((1, D)), full((1, D)), full((1, D)),
            full((D, 2)), full((1, 2)),
            full((HD, H)), full((H, HD)),
        ],
        out_specs=pl.BlockSpec((Bb, 2), lambda i: (i, 0)),
        out_shape=jax.ShapeDtypeStruct((B, 2), _f32),
    )(sel, fsel, res_W, gat_b, g1, b1, Wq, bq, Wk, bk, Wv, bv,
      fc_W, fc_b, g2, b2, out_W, out_b, G8, E8)


def kernel(features, edge_index, url, gat_W, attn_l, attn_r, gat_b, res_W,
           ln1_g, ln1_b, Wq, bq, Wk, bk, Wv, bv, fc_W, fc_b,
           ln2_g, ln2_b, out_W, out_b):
    ids = (url - 1).astype(_i32)
    src = edge_index[0]
    dst = edge_index[1]

    rows = jnp.arange(HD)[:, None]
    cols16 = jnp.arange(16)[None, :]
    blk = (rows // D) == cols16
    Wl = jnp.where(blk, attn_l.reshape(-1)[:, None], 0.0)
    Wr = jnp.where(blk, attn_r.reshape(-1)[:, None], 0.0)
    G8 = jnp.where((rows // D) == jnp.arange(H)[None, :], 1.0, 0.0)
    E8 = G8.T

    feat, er = _tc1(features, gat_W, Wl, Wr)
    asrc, adst, aslot, cnts, rep, fsel = _sc_filter(src, dst, ids, features)
    u = _sc_accumulate(asrc, adst, aslot, cnts, feat, er, ids)

    logits_slot = _tc3(
        u, fsel, res_W, gat_b.reshape(1, HD),
        jnp.tile(ln1_g, H).reshape(1, HD), jnp.tile(ln1_b, H).reshape(1, HD),
        Wq, bq.reshape(1, HD), Wk, bk.reshape(1, HD), Wv, bv.reshape(1, HD),
        fc_W, fc_b.reshape(1, D), ln2_g.reshape(1, D), ln2_b.reshape(1, D),
        out_W, out_b.reshape(1, 2), G8, E8)
    return logits_slot[rep]


# speedup vs baseline: 68.5360x; 1.0060x over previous
import jax
import jax.numpy as jnp
from jax import lax
from jax.experimental import pallas as pl
from jax.experimental.pallas import tpu as pltpu
from jax.experimental.pallas import tpu_sc as plsc

N = 10000
E = 160000
FS = 128
H = 8
D = 64
HD = H * D
B = 1024
FW = HD + 128

NC = 2
NS = 16
NW = NC * NS
EC = E // NW
KMAX = 5072
CH = 64
AR = 5056
OWN = 32
PCAP = 6144
BPW = B // NW

_f32 = jnp.float32
_i32 = jnp.int32


def _tc1_body(x_ref, gw_ref, wl_ref, wr_ref, feat_ref, er_ref):
    f = jnp.dot(x_ref[...], gw_ref[...], preferred_element_type=_f32)
    feat_ref[:, pl.ds(0, HD)] = f
    feat_ref[:, pl.ds(HD, 16)] = jnp.dot(f, wl_ref[...],
                                         preferred_element_type=_f32)
    feat_ref[:, pl.ds(HD + 16, FW - HD - 16)] = jnp.zeros(
        (f.shape[0], FW - HD - 16), _f32)
    er_ref[:, pl.ds(0, 16)] = jnp.dot(f, wr_ref[...],
                                      preferred_element_type=_f32)
    er_ref[:, pl.ds(16, 112)] = jnp.zeros((f.shape[0], 112), _f32)


def _tc1(features, gat_W, Wl, Wr):
    R = 1000
    return pl.pallas_call(
        _tc1_body,
        grid=(N // R,),
        in_specs=[
            pl.BlockSpec((R, FS), lambda i: (i, 0)),
            pl.BlockSpec((FS, HD), lambda i: (0, 0)),
            pl.BlockSpec((HD, 16), lambda i: (0, 0)),
            pl.BlockSpec((HD, 16), lambda i: (0, 0)),
        ],
        out_specs=[
            pl.BlockSpec((R, FW), lambda i: (i, 0)),
            pl.BlockSpec((R, 128), lambda i: (i, 0)),
        ],
        out_shape=[
            jax.ShapeDtypeStruct((N, FW), _f32),
            jax.ShapeDtypeStruct((N, 128), _f32),
        ],
    )(features, gat_W, Wl, Wr)


def _mesh():
    return plsc.VectorSubcoreMesh(
        core_axis_name="c", subcore_axis_name="s",
        num_cores=NC, num_subcores=NS)


_SC_PARAMS = dict(compiler_params=pltpu.CompilerParams(
    needs_layout_passes=False))


def _sc_filter(src, dst, ids, features):
    k = pl.kernel(
        _sc_filter_body,
        out_type=(
            jax.ShapeDtypeStruct((NW * AR,), _i32),
            jax.ShapeDtypeStruct((NW * AR,), _i32),
            jax.ShapeDtypeStruct((NW * AR,), _i32),
            jax.ShapeDtypeStruct((NW * 16,), _i32),
            jax.ShapeDtypeStruct((B,), _i32),
            jax.ShapeDtypeStruct((B, FS), _f32),
        ),
        mesh=_mesh(),
        scratch_types=[
            pltpu.VMEM((N,), _i32),
            pltpu.VMEM((B,), _i32),
            pltpu.VMEM((KMAX,), _i32),
            pltpu.VMEM((KMAX,), _i32),
            pltpu.VMEM((KMAX,), _i32),
            pltpu.VMEM((BPW,), _i32),
            pltpu.VMEM((16,), _i32),
            pltpu.VMEM((BPW, FS), _f32),
        ],
        **_SC_PARAMS,
    )
    return k(src, dst, ids, features)


def _sc_filter_body(src_hbm, dst_hbm, ids_hbm, features_hbm,
                    asrc_hbm, adst_hbm, aslot_hbm, cnt_hbm, rep_hbm, fsel_hbm,
                    pos, idsb, sbuf, dbuf, slist, repb, cntb, fb):
    c = lax.axis_index("c")
    s = lax.axis_index("s")
    w = c * NS + s
    iota16 = jnp.arange(16, dtype=_i32)
    zero16i = jnp.zeros((16,), _i32)

    m1 = jnp.full((16,), -1, _i32)
    def _ip(i, _):
        pos[pl.ds(i * 16, 16)] = m1
        return 0
    lax.fori_loop(0, N // 16, _ip, 0)
    pltpu.sync_copy(ids_hbm, idsb)
    def _sp(j, _):
        idx = idsb[pl.ds(j * 16, 16)]
        plsc.store_scatter(pos, [idx], iota16 + j * 16)
        return 0
    lax.fori_loop(0, B // 16, _sp, 0)

    base = w * EC
    pltpu.sync_copy(src_hbm.at[pl.ds(base, EC)], sbuf.at[pl.ds(0, EC)])
    pltpu.sync_copy(dst_hbm.at[pl.ds(base, EC)], dbuf.at[pl.ds(0, EC)])
    plsc.store_scatter(sbuf, [EC + iota16], zero16i)
    plsc.store_scatter(dbuf, [EC + iota16], zero16i)

    def _flt(i, cnt):
        d16 = dbuf[pl.ds(i * 16, 16)]
        s16 = sbuf[pl.ds(i * 16, 16)]
        p16 = plsc.load_gather(pos, [d16])
        m = (p16 >= 0) & (iota16 < (EC - i * 16))
        plsc.store_compressed(sbuf.at[pl.ds(cnt, 16)], s16, mask=m)
        plsc.store_compressed(dbuf.at[pl.ds(cnt, 16)], d16, mask=m)
        plsc.store_compressed(slist.at[pl.ds(cnt, 16)], p16, mask=m)
        return cnt + jnp.sum(m.astype(_i32))
    kept = lax.fori_loop(0, (EC + 15) // 16, _flt, 0)

    dummy = jnp.full((16,), B, _i32)
    def _pad(j, _):
        idx = kept + j * 16 + iota16
        plsc.store_scatter(sbuf, [idx], zero16i)
        plsc.store_scatter(dbuf, [idx], zero16i)
        plsc.store_scatter(slist, [idx], dummy)
        return 0
    lax.fori_loop(0, CH // 16, _pad, 0)

    def _rep(j, _):
        idx = idsb[pl.ds(w * BPW + j * 16, 16)]
        repb[pl.ds(j * 16, 16)] = plsc.load_gather(pos, [idx])
        return 0
    lax.fori_loop(0, BPW // 16, _rep, 0)
    pltpu.sync_copy(repb, rep_hbm.at[pl.ds(w * BPW, BPW)])

    pltpu.sync_copy(features_hbm.at[idsb.at[pl.ds(w * BPW, BPW)]], fb)
    pltpu.sync_copy(fb, fsel_hbm.at[pl.ds(w * BPW, BPW)])

    padded = lax.shift_left(lax.shift_right_logical(kept + (CH - 1), 6), 6)
    cntb[pl.ds(0, 16)] = jnp.zeros((16,), _i32) + padded
    pltpu.sync_copy(cntb, cnt_hbm.at[pl.ds(w * 16, 16)])
    nchunks = lax.shift_right_logical(padded, 6)
    def _pub(ci, _):
        off = ci * CH
        pltpu.sync_copy(sbuf.at[pl.ds(off, CH)],
                        asrc_hbm.at[pl.ds(w * AR + off, CH)])
        pltpu.sync_copy(dbuf.at[pl.ds(off, CH)],
                        adst_hbm.at[pl.ds(w * AR + off, CH)])
        pltpu.sync_copy(slist.at[pl.ds(off, CH)],
                        aslot_hbm.at[pl.ds(w * AR + off, CH)])
        return 0
    lax.fori_loop(0, nchunks, _pub, 0)


CB = 1024
PCH = 32
RING = 8


def _sc_accumulate(asrc, adst, aslot, cnts, feat, er, ids):
    k = pl.kernel(
        _sc_accumulate_body,
        out_type=jax.ShapeDtypeStruct((B, FW), _f32),
        mesh=_mesh(),
        scratch_types=[
            pltpu.VMEM((NW * 16,), _i32),
            pltpu.VMEM((RING * CB,), _i32),
            pltpu.VMEM((RING * CB,), _i32),
            pltpu.VMEM((RING * CB,), _i32),
            pltpu.VMEM((PCAP,), _i32),
            pltpu.VMEM((PCAP,), _i32),
            pltpu.VMEM((PCAP,), _i32),
            pltpu.VMEM((2 * PCH, FW), _f32),
            pltpu.VMEM((OWN + 8, 128), _f32),
            pltpu.VMEM((BPW,), _i32),
            pltpu.VMEM((OWN + 8, FW), _f32),
            pltpu.SemaphoreType.DMA,
            pltpu.SemaphoreType.DMA,
        ],
        **_SC_PARAMS,
    )
    return k(asrc, adst, aslot, cnts, feat, er, ids)


def _sc_accumulate_body(asrc_hbm, adst_hbm, aslot_hbm, cnt_hbm,
                        feat_hbm, er_hbm, ids_hbm, u_hbm,
                        cbuf, tsrc, tdst, tslot, psrc, pdst, pslot,
                        featb, er_loc, idsb, u_loc, asem, gsem):
    c = lax.axis_index("c")
    s = lax.axis_index("s")
    w = c * NS + s
    iota16 = jnp.arange(16, dtype=_i32)
    zero16f = jnp.zeros((16,), _f32)
    zero16i = jnp.zeros((16,), _i32)

    @plsc.parallel_loop(0, OWN + 8, 1, unroll=2)
    def _z(j):
        for q in range(FW // 16):
            u_loc[j, pl.ds(q * 16, 16)] = zero16f
        er_loc[j, pl.ds(0, 16)] = zero16f

    pltpu.sync_copy(cnt_hbm, cbuf)

    pltpu.sync_copy(ids_hbm.at[pl.ds(w * OWN, OWN)], idsb)
    pltpu.sync_copy(er_hbm.at[idsb], er_loc.at[pl.ds(0, OWN)])

    def _gissue(off, parity):
        pltpu.async_copy(feat_hbm.at[psrc.at[pl.ds(off, PCH)]],
                         featb.at[pl.ds(parity * PCH, PCH)], gsem)

    def _gwait(parity):
        pltpu.make_async_copy(feat_hbm.at[pl.ds(0, PCH)],
                              featb.at[pl.ds(parity * PCH, PCH)], gsem).wait()

    def _compute(off, parity):
        base = parity * PCH
        @plsc.parallel_loop(0, PCH // 16, 1)
        def _acc(g):
            row16 = pslot[pl.ds(off + g * 16, 16)]
            for k in range(16):
                j = base + g * 16 + k
                r = row16[k]
                e = featb[j, pl.ds(HD, 16)] + er_loc[r, pl.ds(0, 16)]
                e = jnp.where(e > 0.0, e, 0.2 * e)
                eerow = jnp.exp(e)
                for h in range(H):
                    sc = eerow[h]
                    for q in range(D // 16):
                        sl = pl.ds(h * D + q * 16, 16)
                        plsc.addupdate(u_loc.at[r, sl], featb[j, sl] * sc)
                plsc.addupdate(u_loc.at[r, pl.ds(HD, 16)], eerow)

    def _run(nproc):
        @pl.when(nproc > 0)
        def _():
            _gissue(0, 0)
            def _p(i, _):
                parity = lax.rem(i, 2)
                _gwait(parity)
                @pl.when(i + 1 < nproc)
                def _():
                    _gissue((i + 1) * PCH, 1 - parity)
                _compute(i * PCH, parity)
                return 0
            lax.fori_loop(0, nproc, _p, 0)

    def _issue(t, parity):
        pltpu.async_copy(asrc_hbm.at[pl.ds(t * AR, CB)],
                         tsrc.at[pl.ds(parity * CB, CB)], asem)
        pltpu.async_copy(adst_hbm.at[pl.ds(t * AR, CB)],
                         tdst.at[pl.ds(parity * CB, CB)], asem)
        pltpu.async_copy(aslot_hbm.at[pl.ds(t * AR, CB)],
                         tslot.at[pl.ds(parity * CB, CB)], asem)

    def _wait(parity):
        pltpu.make_async_copy(asrc_hbm.at[pl.ds(0, CB)],
                              tsrc.at[pl.ds(parity * CB, CB)], asem).wait()
        pltpu.make_async_copy(adst_hbm.at[pl.ds(0, CB)],
                              tdst.at[pl.ds(parity * CB, CB)], asem).wait()
        pltpu.make_async_copy(aslot_hbm.at[pl.ds(0, CB)],
                              tslot.at[pl.ds(parity * CB, CB)], asem).wait()

    def _filter_append(boff, lim, lo, pend):
        ngrp = lax.shift_right_logical(
            jnp.minimum(lim - lo, CB) + 15, 4)
        @plsc.parallel_loop(0, ngrp, 1, unroll=2, carry=pend)
        def _grp(g, pend):
            sl16 = tslot[pl.ds(boff + g * 16, 16)]
            valid = (lo + g * 16 + iota16) < lim
            m = (lax.shift_right_logical(sl16, 5) == w) & valid
            plsc.store_compressed(psrc.at[pl.ds(pend, 16)],
                                  tsrc[pl.ds(boff + g * 16, 16)], mask=m)
            plsc.store_compressed(pdst.at[pl.ds(pend, 16)],
                                  tdst[pl.ds(boff + g * 16, 16)], mask=m)
            plsc.store_compressed(pslot.at[pl.ds(pend, 16)],
                                  sl16 - OWN * w, mask=m)
            return pend + jnp.sum(m.astype(_i32))
        return _grp

    for r in range(RING):
        _issue(r, r)

    def _tile(t, pend):
        slot = lax.rem(t, RING)
        _wait(slot)
        cnt = cbuf[pl.ds(t * 16, 16)][0]
        pend = _filter_append(slot * CB, cnt, 0, pend)
        nex = lax.shift_right_logical(
            jnp.maximum(cnt, CB) - CB + (CB - 1), 10)
        def _extra(ec, pend):
            aoff = t * AR + CB + ec * CB
            pltpu.sync_copy(asrc_hbm.at[pl.ds(aoff, CB)],
                            tsrc.at[pl.ds(slot * CB, CB)])
            pltpu.sync_copy(adst_hbm.at[pl.ds(aoff, CB)],
                            tdst.at[pl.ds(slot * CB, CB)])
            pltpu.sync_copy(aslot_hbm.at[pl.ds(aoff, CB)],
                            tslot.at[pl.ds(slot * CB, CB)])
            return _filter_append(slot * CB, cnt, CB + ec * CB, pend)
        pend = lax.fori_loop(0, nex, _extra, pend)
        @pl.when(t + RING < NW)
        def _():
            _issue(t + RING, slot)
        nd = jnp.where(pend >= PCAP - AR,
                       lax.shift_right_logical(pend, 5), 0)
        _run(nd)
        rem_base = nd * PCH
        for k in range(PCH // 16):
            v0 = psrc[pl.ds(rem_base + k * 16, 16)]
            v1 = pdst[pl.ds(rem_base + k * 16, 16)]
            v2 = pslot[pl.ds(rem_base + k * 16, 16)]
            psrc[pl.ds(k * 16, 16)] = v0
            pdst[pl.ds(k * 16, 16)] = v1
            pslot[pl.ds(k * 16, 16)] = v2
        return pend - rem_base

    pend = lax.fori_loop(0, NW, _tile, 0)

    trash = jnp.full((16,), OWN, _i32)
    def _pad(j, _):
        idx = pend + j * 16 + iota16
        plsc.store_scatter(psrc, [idx], zero16i)
        plsc.store_scatter(pdst, [idx], zero16i)
        plsc.store_scatter(pslot, [idx], trash)
        return 0
    lax.fori_loop(0, PCH // 16, _pad, 0)
    _run(lax.shift_right_logical(pend + (PCH - 1), 5))

    pltpu.sync_copy(u_loc.at[pl.ds(0, OWN)], u_hbm.at[pl.ds(w * OWN, OWN)])


def _tc3_body(sel_ref, fsel_ref, rw_ref, gb_ref, g1_ref, b1_ref,
              wq_ref, bq_ref, wk_ref, bk_ref, wv_ref, bv_ref,
              fw_ref, fb_ref, g2_ref, b2_ref, ow_ref, ob_ref,
              g8_ref, e8_ref, out_ref):
    g8 = g8_ref[...]
    e8 = e8_ref[...]
    den = jnp.dot(sel_ref[:, pl.ds(HD, H)], e8,
                  preferred_element_type=_f32) + 1e-9
    agg = sel_ref[:, pl.ds(0, HD)] / den
    x0 = agg + jnp.dot(fsel_ref[...], rw_ref[...],
                       preferred_element_type=_f32) + gb_ref[...]
    g8n = g8 * (1.0 / D)
    mu = jnp.dot(x0, g8n, preferred_element_type=_f32)
    xc = x0 - jnp.dot(mu, e8, preferred_element_type=_f32)
    var = jnp.dot(xc * xc, g8n, preferred_element_type=_f32)
    inv = jax.lax.rsqrt(var + 1e-6)
    x = xc * jnp.dot(inv, e8, preferred_element_type=_f32) * g1_ref[...] + b1_ref[...]

    qs, ks, vs, rsd = [], [], [], []
    for l in range(H):
        xl = x[:, l * D:(l + 1) * D]
        rsd.append(xl)
        qs.append(jnp.dot(xl, wq_ref[...], preferred_element_type=_f32) + bq_ref[...])
        ks.append(jnp.dot(xl, wk_ref[...], preferred_element_type=_f32) + bk_ref[...])
        vs.append(jnp.dot(xl, wv_ref[...], preferred_element_type=_f32) + bv_ref[...])

    acc = None
    scale = 1.0 / (D ** 0.5)
    for l in range(H):
        s_lm = [jnp.dot(qs[l] * ks[m], g8, preferred_element_type=_f32) * scale
                for m in range(H)]
        mx = s_lm[0]
        for m in range(1, H):
            mx = jnp.maximum(mx, s_lm[m])
        ex = [jnp.exp(sv - mx) for sv in s_lm]
        ssum = ex[0]
        for m in range(1, H):
            ssum = ssum + ex[m]
        rs = 1.0 / ssum
        o_l = None
        for m in range(H):
            t = jnp.dot(ex[m] * rs, e8, preferred_element_type=_f32) * vs[m]
            o_l = t if o_l is None else o_l + t
        f_l = jnp.dot(o_l, fw_ref[...], preferred_element_type=_f32) + fb_ref[...] + rsd[l]
        mu2 = jnp.mean(f_l, axis=1, keepdims=True)
        d2 = f_l - mu2
        var2 = jnp.mean(d2 * d2, axis=1, keepdims=True)
        z_l = d2 * jax.lax.rsqrt(var2 + 1e-5) * g2_ref[...] + b2_ref[...]
        acc = z_l if acc is None else acc + z_l
    out_ref[...] = jnp.dot(acc, ow_ref[...], preferred_element_type=_f32) + ob_ref[...]


def _tc3(sel, fsel, res_W, gat_b, g1, b1, Wq, bq, Wk, bk, Wv, bv,
         fc_W, fc_b, g2, b2, out_W, out_b, G8, E8):
    Bb = 128
    full = lambda shape: pl.BlockSpec(shape, lambda i: tuple(0 for _ in shape))
    return pl.pallas_call(
        _tc3_body,
        grid=(B // Bb,),
        in_specs=[
            pl.BlockSpec((Bb, FW), lambda i: (i, 0)),
            pl.BlockSpec((Bb, FS), lambda i: (i, 0)),
            full((FS, HD)), full((1, HD)), full((1, HD)), full((1, HD)),
            full((D, HD)), full((1, HD)), full((D, HD)), full((1, HD)),
            full((D, HD)), full((1, HD)),
            full((HD, D)), full((1, D)), full((1, D)), full((1, D)),
            full((D, 2)), full((1, 2)),
            full((HD, H)), full((H, HD)),
        ],
        out_specs=pl.BlockSpec((Bb, 2), lambda i: (i, 0)),
        out_shape=jax.ShapeDtypeStruct((B, 2), _f32),
    )(sel, fsel, res_W, gat_b, g1, b1, Wq, bq, Wk, bk, Wv, bv,
      fc_W, fc_b, g2, b2, out_W, out_b, G8, E8)


def kernel(features, edge_index, url, gat_W, attn_l, attn_r, gat_b, res_W,
           ln1_g, ln1_b, Wq, bq, Wk, bk, Wv, bv, fc_W, fc_b,
           ln2_g, ln2_b, out_W, out_b):
    ids = (url - 1).astype(_i32)
    src = edge_index[0]
    dst = edge_index[1]

    rows = jnp.arange(HD)[:, None]
    cols16 = jnp.arange(16)[None, :]
    blk = (rows // D) == cols16
    Wl = jnp.where(blk, attn_l.reshape(-1)[:, None], 0.0)
    Wr = jnp.where(blk, attn_r.reshape(-1)[:, None], 0.0)
    G8 = jnp.where((rows // D) == jnp.arange(H)[None, :], 1.0, 0.0)
    E8 = G8.T

    feat, er = _tc1(features, gat_W, Wl, Wr)
    asrc, adst, aslot, cnts, rep, fsel = _sc_filter(src, dst, ids, features)
    u = _sc_accumulate(asrc, adst, aslot, cnts, feat, er, ids)

    logits_slot = _tc3(
        u, fsel, res_W, gat_b.reshape(1, HD),
        jnp.tile(ln1_g, H).reshape(1, HD), jnp.tile(ln1_b, H).reshape(1, HD),
        Wq, bq.reshape(1, HD), Wk, bk.reshape(1, HD), Wv, bv.reshape(1, HD),
        fc_W, fc_b.reshape(1, D), ln2_g.reshape(1, D), ln2_b.reshape(1, D),
        out_W, out_b.reshape(1, 2), G8, E8)
    return logits_slot[rep]


# speedup vs baseline: 73.6615x; 1.0748x over previous
import jax
import jax.numpy as jnp
from jax import lax
from jax.experimental import pallas as pl
from jax.experimental.pallas import tpu as pltpu
from jax.experimental.pallas import tpu_sc as plsc

N = 10000
E = 160000
FS = 128
H = 8
D = 64
HD = H * D
B = 1024
FW = HD + 128

NC = 2
NS = 16
NW = NC * NS
EC = E // NW
KMAX = 5072
CH = 64
AR = 5056
OWN = 32
PCAP = 6144
BPW = B // NW

_f32 = jnp.float32
_i32 = jnp.int32


def _tc1_body(x_ref, gw_ref, wl_ref, wr_ref, feat_ref, er_ref):
    f = jnp.dot(x_ref[...], gw_ref[...], preferred_element_type=_f32)
    feat_ref[:, pl.ds(0, HD)] = f
    feat_ref[:, pl.ds(HD, 16)] = jnp.dot(f, wl_ref[...],
                                         preferred_element_type=_f32)
    feat_ref[:, pl.ds(HD + 16, FW - HD - 16)] = jnp.zeros(
        (f.shape[0], FW - HD - 16), _f32)
    er_ref[:, pl.ds(0, 16)] = jnp.dot(f, wr_ref[...],
                                      preferred_element_type=_f32)


def _tc1(features, gat_W, Wl, Wr):
    R = 1000
    return pl.pallas_call(
        _tc1_body,
        grid=(N // R,),
        in_specs=[
            pl.BlockSpec((R, FS), lambda i: (i, 0)),
            pl.BlockSpec((FS, HD), lambda i: (0, 0)),
            pl.BlockSpec((HD, 16), lambda i: (0, 0)),
            pl.BlockSpec((HD, 16), lambda i: (0, 0)),
        ],
        out_specs=[
            pl.BlockSpec((R, FW), lambda i: (i, 0)),
            pl.BlockSpec((R, 128), lambda i: (i, 0)),
        ],
        out_shape=[
            jax.ShapeDtypeStruct((N, FW), _f32),
            jax.ShapeDtypeStruct((N, 128), _f32),
        ],
    )(features, gat_W, Wl, Wr)


def _mesh():
    return plsc.VectorSubcoreMesh(
        core_axis_name="c", subcore_axis_name="s",
        num_cores=NC, num_subcores=NS)


_SC_PARAMS = dict(compiler_params=pltpu.CompilerParams(
    needs_layout_passes=False))


def _sc_filter(src, dst, ids, features):
    k = pl.kernel(
        _sc_filter_body,
        out_type=(
            jax.ShapeDtypeStruct((NW * AR,), _i32),
            jax.ShapeDtypeStruct((NW * AR,), _i32),
            jax.ShapeDtypeStruct((NW * AR,), _i32),
            jax.ShapeDtypeStruct((NW * 16,), _i32),
            jax.ShapeDtypeStruct((B,), _i32),
            jax.ShapeDtypeStruct((B, FS), _f32),
        ),
        mesh=_mesh(),
        scratch_types=[
            pltpu.VMEM((N,), _i32),
            pltpu.VMEM((B,), _i32),
            pltpu.VMEM((KMAX,), _i32),
            pltpu.VMEM((KMAX,), _i32),
            pltpu.VMEM((KMAX,), _i32),
            pltpu.VMEM((BPW,), _i32),
            pltpu.VMEM((16,), _i32),
            pltpu.VMEM((BPW, FS), _f32),
        ],
        **_SC_PARAMS,
    )
    return k(src, dst, ids, features)


def _sc_filter_body(src_hbm, dst_hbm, ids_hbm, features_hbm,
                    asrc_hbm, adst_hbm, aslot_hbm, cnt_hbm, rep_hbm, fsel_hbm,
                    pos, idsb, sbuf, dbuf, slist, repb, cntb, fb):
    c = lax.axis_index("c")
    s = lax.axis_index("s")
    w = c * NS + s
    iota16 = jnp.arange(16, dtype=_i32)
    zero16i = jnp.zeros((16,), _i32)

    m1 = jnp.full((16,), -1, _i32)
    def _ip(i, _):
        pos[pl.ds(i * 16, 16)] = m1
        return 0
    lax.fori_loop(0, N // 16, _ip, 0)
    pltpu.sync_copy(ids_hbm, idsb)
    def _sp(j, _):
        idx = idsb[pl.ds(j * 16, 16)]
        plsc.store_scatter(pos, [idx], iota16 + j * 16)
        return 0
    lax.fori_loop(0, B // 16, _sp, 0)

    base = w * EC
    pltpu.sync_copy(src_hbm.at[pl.ds(base, EC)], sbuf.at[pl.ds(0, EC)])
    pltpu.sync_copy(dst_hbm.at[pl.ds(base, EC)], dbuf.at[pl.ds(0, EC)])
    plsc.store_scatter(sbuf, [EC + iota16], zero16i)
    plsc.store_scatter(dbuf, [EC + iota16], zero16i)

    def _flt(i, cnt):
        d16 = dbuf[pl.ds(i * 16, 16)]
        s16 = sbuf[pl.ds(i * 16, 16)]
        p16 = plsc.load_gather(pos, [d16])
        m = (p16 >= 0) & (iota16 < (EC - i * 16))
        plsc.store_compressed(sbuf.at[pl.ds(cnt, 16)], s16, mask=m)
        plsc.store_compressed(dbuf.at[pl.ds(cnt, 16)], d16, mask=m)
        plsc.store_compressed(slist.at[pl.ds(cnt, 16)], p16, mask=m)
        return cnt + jnp.sum(m.astype(_i32))
    kept = lax.fori_loop(0, (EC + 15) // 16, _flt, 0)

    dummy = jnp.full((16,), B, _i32)
    def _pad(j, _):
        idx = kept + j * 16 + iota16
        plsc.store_scatter(sbuf, [idx], zero16i)
        plsc.store_scatter(dbuf, [idx], zero16i)
        plsc.store_scatter(slist, [idx], dummy)
        return 0
    lax.fori_loop(0, CH // 16, _pad, 0)

    def _rep(j, _):
        idx = idsb[pl.ds(w * BPW + j * 16, 16)]
        repb[pl.ds(j * 16, 16)] = plsc.load_gather(pos, [idx])
        return 0
    lax.fori_loop(0, BPW // 16, _rep, 0)
    pltpu.sync_copy(repb, rep_hbm.at[pl.ds(w * BPW, BPW)])

    pltpu.sync_copy(features_hbm.at[idsb.at[pl.ds(w * BPW, BPW)]], fb)
    pltpu.sync_copy(fb, fsel_hbm.at[pl.ds(w * BPW, BPW)])

    padded = lax.shift_left(lax.shift_right_logical(kept + (CH - 1), 6), 6)
    cntb[pl.ds(0, 16)] = jnp.zeros((16,), _i32) + padded
    pltpu.sync_copy(cntb, cnt_hbm.at[pl.ds(w * 16, 16)])
    nchunks = lax.shift_right_logical(padded, 6)
    def _pub(ci, _):
        off = ci * CH
        pltpu.sync_copy(sbuf.at[pl.ds(off, CH)],
                        asrc_hbm.at[pl.ds(w * AR + off, CH)])
        pltpu.sync_copy(dbuf.at[pl.ds(off, CH)],
                        adst_hbm.at[pl.ds(w * AR + off, CH)])
        pltpu.sync_copy(slist.at[pl.ds(off, CH)],
                        aslot_hbm.at[pl.ds(w * AR + off, CH)])
        return 0
    lax.fori_loop(0, nchunks, _pub, 0)


CB = 1024
PCH = 32
RING = 8


def _sc_accumulate(asrc, adst, aslot, cnts, feat, er, ids):
    k = pl.kernel(
        _sc_accumulate_body,
        out_type=jax.ShapeDtypeStruct((B, FW), _f32),
        mesh=_mesh(),
        scratch_types=[
            pltpu.VMEM((NW * 16,), _i32),
            pltpu.VMEM((RING * CB,), _i32),
            pltpu.VMEM((RING * CB,), _i32),
            pltpu.VMEM((RING * CB,), _i32),
            pltpu.VMEM((PCAP,), _i32),
            pltpu.VMEM((PCAP,), _i32),
            pltpu.VMEM((PCAP,), _i32),
            pltpu.VMEM((2 * PCH, FW), _f32),
            pltpu.VMEM((OWN + 8, 128), _f32),
            pltpu.VMEM((BPW,), _i32),
            pltpu.VMEM((OWN + 8, FW), _f32),
            pltpu.SemaphoreType.DMA,
            pltpu.SemaphoreType.DMA,
        ],
        **_SC_PARAMS,
    )
    return k(asrc, adst, aslot, cnts, feat, er, ids)


def _sc_accumulate_body(asrc_hbm, adst_hbm, aslot_hbm, cnt_hbm,
                        feat_hbm, er_hbm, ids_hbm, u_hbm,
                        cbuf, tsrc, tdst, tslot, psrc, pdst, pslot,
                        featb, er_loc, idsb, u_loc, asem, gsem):
    c = lax.axis_index("c")
    s = lax.axis_index("s")
    w = c * NS + s
    iota16 = jnp.arange(16, dtype=_i32)
    zero16f = jnp.zeros((16,), _f32)
    zero16i = jnp.zeros((16,), _i32)

    @plsc.parallel_loop(0, OWN + 8, 1, unroll=2)
    def _z(j):
        for q in range(FW // 16):
            u_loc[j, pl.ds(q * 16, 16)] = zero16f
        er_loc[j, pl.ds(0, 16)] = zero16f

    pltpu.sync_copy(cnt_hbm, cbuf)

    pltpu.sync_copy(ids_hbm.at[pl.ds(w * OWN, OWN)], idsb)
    pltpu.sync_copy(er_hbm.at[idsb], er_loc.at[pl.ds(0, OWN)])

    def _gissue(off, parity):
        pltpu.async_copy(feat_hbm.at[psrc.at[pl.ds(off, PCH)]],
                         featb.at[pl.ds(parity * PCH, PCH)], gsem)

    def _gwait(parity):
        pltpu.make_async_copy(feat_hbm.at[pl.ds(0, PCH)],
                              featb.at[pl.ds(parity * PCH, PCH)], gsem).wait()

    def _compute(off, parity):
        base = parity * PCH
        @plsc.parallel_loop(0, PCH // 16, 1)
        def _acc(g):
            row16 = pslot[pl.ds(off + g * 16, 16)]
            for k in range(16):
                j = base + g * 16 + k
                r = row16[k]
                e = featb[j, pl.ds(HD, 16)] + er_loc[r, pl.ds(0, 16)]
                e = jnp.where(e > 0.0, e, 0.2 * e)
                eerow = jnp.exp(e)
                for h in range(H):
                    sc = eerow[h]
                    for q in range(D // 16):
                        sl = pl.ds(h * D + q * 16, 16)
                        plsc.addupdate(u_loc.at[r, sl], featb[j, sl] * sc)
                plsc.addupdate(u_loc.at[r, pl.ds(HD, 16)], eerow)

    def _run(nproc):
        @pl.when(nproc > 0)
        def _():
            _gissue(0, 0)
            def _p(i, _):
                parity = lax.rem(i, 2)
                _gwait(parity)
                @pl.when(i + 1 < nproc)
                def _():
                    _gissue((i + 1) * PCH, 1 - parity)
                _compute(i * PCH, parity)
                return 0
            lax.fori_loop(0, nproc, _p, 0)

    def _issue(t, parity):
        pltpu.async_copy(asrc_hbm.at[pl.ds(t * AR, CB)],
                         tsrc.at[pl.ds(parity * CB, CB)], asem)
        pltpu.async_copy(adst_hbm.at[pl.ds(t * AR, CB)],
                         tdst.at[pl.ds(parity * CB, CB)], asem)
        pltpu.async_copy(aslot_hbm.at[pl.ds(t * AR, CB)],
                         tslot.at[pl.ds(parity * CB, CB)], asem)

    def _wait(parity):
        pltpu.make_async_copy(asrc_hbm.at[pl.ds(0, CB)],
                              tsrc.at[pl.ds(parity * CB, CB)], asem).wait()
        pltpu.make_async_copy(adst_hbm.at[pl.ds(0, CB)],
                              tdst.at[pl.ds(parity * CB, CB)], asem).wait()
        pltpu.make_async_copy(aslot_hbm.at[pl.ds(0, CB)],
                              tslot.at[pl.ds(parity * CB, CB)], asem).wait()

    def _filter_append(boff, lim, lo, pend):
        ngrp = lax.shift_right_logical(
            jnp.minimum(lim - lo, CB) + 15, 4)
        @plsc.parallel_loop(0, ngrp, 1, unroll=2, carry=pend)
        def _grp(g, pend):
            sl16 = tslot[pl.ds(boff + g * 16, 16)]
            valid = (lo + g * 16 + iota16) < lim
            m = (lax.shift_right_logical(sl16, 5) == w) & valid
            plsc.store_compressed(psrc.at[pl.ds(pend, 16)],
                                  tsrc[pl.ds(boff + g * 16, 16)], mask=m)
            plsc.store_compressed(pdst.at[pl.ds(pend, 16)],
                                  tdst[pl.ds(boff + g * 16, 16)], mask=m)
            plsc.store_compressed(pslot.at[pl.ds(pend, 16)],
                                  sl16 - OWN * w, mask=m)
            return pend + jnp.sum(m.astype(_i32))
        return _grp

    for r in range(RING):
        _issue(r, r)

    def _tile(t, pend):
        slot = lax.rem(t, RING)
        _wait(slot)
        cnt = cbuf[pl.ds(t * 16, 16)][0]
        pend = _filter_append(slot * CB, cnt, 0, pend)
        nex = lax.shift_right_logical(
            jnp.maximum(cnt, CB) - CB + (CB - 1), 10)
        def _extra(ec, pend):
            aoff = t * AR + CB + ec * CB
            pltpu.sync_copy(asrc_hbm.at[pl.ds(aoff, CB)],
                            tsrc.at[pl.ds(slot * CB, CB)])
            pltpu.sync_copy(adst_hbm.at[pl.ds(aoff, CB)],
                            tdst.at[pl.ds(slot * CB, CB)])
            pltpu.sync_copy(aslot_hbm.at[pl.ds(aoff, CB)],
                            tslot.at[pl.ds(slot * CB, CB)])
            return _filter_append(slot * CB, cnt, CB + ec * CB, pend)
        pend = lax.fori_loop(0, nex, _extra, pend)
        @pl.when(t + RING < NW)
        def _():
            _issue(t + RING, slot)
        nd = jnp.where(pend >= PCAP - AR,
                       lax.shift_right_logical(pend, 5), 0)
        _run(nd)
        rem_base = nd * PCH
        for k in range(PCH // 16):
            v0 = psrc[pl.ds(rem_base + k * 16, 16)]
            v1 = pdst[pl.ds(rem_base + k * 16, 16)]
            v2 = pslot[pl.ds(rem_base + k * 16, 16)]
            psrc[pl.ds(k * 16, 16)] = v0
            pdst[pl.ds(k * 16, 16)] = v1
            pslot[pl.ds(k * 16, 16)] = v2
        return pend - rem_base

    pend = lax.fori_loop(0, NW, _tile, 0)

    trash = jnp.full((16,), OWN, _i32)
    def _pad(j, _):
        idx = pend + j * 16 + iota16
        plsc.store_scatter(psrc, [idx], zero16i)
        plsc.store_scatter(pdst, [idx], zero16i)
        plsc.store_scatter(pslot, [idx], trash)
        return 0
    lax.fori_loop(0, PCH // 16, _pad, 0)
    _run(lax.shift_right_logical(pend + (PCH - 1), 5))

    pltpu.sync_copy(u_loc.at[pl.ds(0, OWN)], u_hbm.at[pl.ds(w * OWN, OWN)])


def _tc3_body(sel_ref, fsel_ref, rw_ref, gb_ref, g1_ref, b1_ref,
              wqkv_ref, bqkv_ref,
              fw_ref, fb_ref, g2_ref, b2_ref, ow_ref, ob_ref,
              g8_ref, e8_ref, out_ref):
    g8 = g8_ref[...]
    e8 = e8_ref[...]
    den = jnp.dot(sel_ref[:, pl.ds(HD, H)], e8,
                  preferred_element_type=_f32) + 1e-9
    agg = sel_ref[:, pl.ds(0, HD)] / den
    x0 = agg + jnp.dot(fsel_ref[...], rw_ref[...],
                       preferred_element_type=_f32) + gb_ref[...]
    g8n = g8 * (1.0 / D)
    mu = jnp.dot(x0, g8n, preferred_element_type=_f32)
    xc = x0 - jnp.dot(mu, e8, preferred_element_type=_f32)
    var = jnp.dot(xc * xc, g8n, preferred_element_type=_f32)
    inv = jax.lax.rsqrt(var + 1e-6)
    x = xc * jnp.dot(inv, e8, preferred_element_type=_f32) * g1_ref[...] + b1_ref[...]

    qs, ks, vs, rsd = [], [], [], []
    for l in range(H):
        xl = x[:, l * D:(l + 1) * D]
        rsd.append(xl)
        qkv = jnp.dot(xl, wqkv_ref[...], preferred_element_type=_f32) + bqkv_ref[...]
        qs.append(qkv[:, 0:HD])
        ks.append(qkv[:, HD:2 * HD])
        vs.append(qkv[:, 2 * HD:3 * HD])

    acc = None
    scale = 1.0 / (D ** 0.5)
    for l in range(H):
        ex = [jnp.exp(jnp.dot(qs[l] * ks[m], g8,
                              preferred_element_type=_f32) * scale)
              for m in range(H)]
        ssum = ex[0]
        for m in range(1, H):
            ssum = ssum + ex[m]
        rs = 1.0 / ssum
        o_l = None
        for m in range(H):
            t = jnp.dot(ex[m] * rs, e8, preferred_element_type=_f32) * vs[m]
            o_l = t if o_l is None else o_l + t
        f_l = jnp.dot(o_l, fw_ref[...], preferred_element_type=_f32) + fb_ref[...] + rsd[l]
        mu2 = jnp.mean(f_l, axis=1, keepdims=True)
        d2 = f_l - mu2
        var2 = jnp.mean(d2 * d2, axis=1, keepdims=True)
        z_l = d2 * jax.lax.rsqrt(var2 + 1e-5) * g2_ref[...] + b2_ref[...]
        acc = z_l if acc is None else acc + z_l
    out_ref[...] = jnp.dot(acc, ow_ref[...], preferred_element_type=_f32) + ob_ref[...]


def _tc3(sel, fsel, res_W, gat_b, g1, b1, Wqkv, bqkv,
         fc_W, fc_b, g2, b2, out_W, out_b, G8, E8):
    Bb = 256
    full = lambda shape: pl.BlockSpec(shape, lambda i: tuple(0 for _ in shape))
    return pl.pallas_call(
        _tc3_body,
        grid=(B // Bb,),
        in_specs=[
            pl.BlockSpec((Bb, FW), lambda i: (i, 0)),
            pl.BlockSpec((Bb, FS), lambda i: (i, 0)),
            full((FS, HD)), full((1, HD)), full((1, HD)), full((1, HD)),
            full((D, 3 * HD)), full((1, 3 * HD)),
            full((HD, D)), full((1, D)), full((1, D)), full((1, D)),
            full((D, 2)), full((1, 2)),
            full((HD, H)), full((H, HD)),
        ],
        out_specs=pl.BlockSpec((Bb, 2), lambda i: (i, 0)),
        out_shape=jax.ShapeDtypeStruct((B, 2), _f32),
    )(sel, fsel, res_W, gat_b, g1, b1, Wqkv, bqkv,
      fc_W, fc_b, g2, b2, out_W, out_b, G8, E8)


def kernel(features, edge_index, url, gat_W, attn_l, attn_r, gat_b, res_W,
           ln1_g, ln1_b, Wq, bq, Wk, bk, Wv, bv, fc_W, fc_b,
           ln2_g, ln2_b, out_W, out_b):
    ids = (url - 1).astype(_i32)
    src = edge_index[0]
    dst = edge_index[1]

    rows = jnp.arange(HD)[:, None]
    cols16 = jnp.arange(16)[None, :]
    blk = (rows // D) == cols16
    Wl = jnp.where(blk, attn_l.reshape(-1)[:, None], 0.0)
    Wr = jnp.where(blk, attn_r.reshape(-1)[:, None], 0.0)
    G8 = jnp.where((rows // D) == jnp.arange(H)[None, :], 1.0, 0.0)
    E8 = G8.T

    feat, er = _tc1(features, gat_W, Wl, Wr)
    asrc, adst, aslot, cnts, rep, fsel = _sc_filter(src, dst, ids, features)
    u = _sc_accumulate(asrc, adst, aslot, cnts, feat, er, ids)

    Wqkv = jnp.concatenate([Wq, Wk, Wv], axis=1)
    bqkv = jnp.concatenate([bq, bk, bv]).reshape(1, 3 * HD)
    logits_slot = _tc3(
        u, fsel, res_W, gat_b.reshape(1, HD),
        jnp.tile(ln1_g, H).reshape(1, HD), jnp.tile(ln1_b, H).reshape(1, HD),
        Wqkv, bqkv,
        fc_W, fc_b.reshape(1, D), ln2_g.reshape(1, D), ln2_b.reshape(1, D),
        out_W, out_b.reshape(1, 2), G8, E8)
    return logits_slot[rep]
